# baseline (speedup 1.0000x reference)
import functools

import jax
import jax.numpy as jnp
from jax import lax
from jax.experimental import pallas as pl
from jax.experimental.pallas import tpu as pltpu
from jax.experimental.pallas import tpu_sc as plsc

N = 50000
D = 128
K = 10
NCLS = 16

NC = 2
NS = 16
NW = NC * NS
NPAD = 50176
NODES_W = NPAD // NW
EDGES_W = NODES_W * K

A1_CH = 112
A1_IT = EDGES_W // A1_CH

CH_N = 16
CH_E = CH_N * K
CH_IT = NODES_W // CH_N

_mesh = functools.partial(plsc.VectorSubcoreMesh,
                          core_axis_name="c", subcore_axis_name="s")


def _wid():
    return lax.axis_index("c") * NS + lax.axis_index("s")


def _a1_body(src_hbm, deg2_hbm, i0, i1, i2, i3, ones_v, zslice_v, deg_sh,
             si0, si1, si2, si3, ss0, ss1, ss2, ss3):
    c = lax.axis_index("c")
    s = lax.axis_index("s")
    w = _wid()
    idx = [i0, i1, i2, i3]
    semi = [si0, si1, si2, si3]
    sems = [ss0, ss1, ss2, ss3]
    zero16 = jnp.zeros((16,), jnp.float32)
    for j in range(A1_CH // 16):
        ones_v[pl.ds(j * 16, 16)] = zero16 + 1.0

    slice_sz = NPAD // NS

    def zbody(i, _):
        zslice_v[pl.ds(i * 16, 16)] = zero16
        return 0

    lax.fori_loop(0, slice_sz // 16, zbody, 0)
    pltpu.sync_copy(zslice_v, deg_sh.at[pl.ds(s * slice_sz, slice_sz)])
    plsc.subcore_barrier()

    def pf(cc, u):
        pltpu.async_copy(src_hbm.at[pl.ds(w * EDGES_W + cc * A1_CH, A1_CH)],
                         idx[u], semi[u])

    def wait_idx(u):
        pltpu.make_async_copy(src_hbm.at[pl.ds(0, A1_CH)], idx[u],
                              semi[u]).wait()

    def wait_sc(u):
        pltpu.make_async_copy(src_hbm.at[pl.ds(0, A1_CH)], ones_v,
                              sems[u]).wait()

    pf(0, 0)
    pf(1, 1)

    def body(c4, _):
        for u in range(4):
            cc = c4 * 4 + u

            @pl.when(cc >= 2)
            def _():
                wait_sc((u + 2) % 4)

            @pl.when(cc + 2 < A1_IT)
            def _():
                pf(cc + 2, (u + 2) % 4)

            wait_idx(u)
            pltpu.async_copy(ones_v, deg_sh.at[idx[u]], sems[u], add=True)
        return 0

    lax.fori_loop(0, A1_IT // 4, body, 0)
    wait_sc((A1_IT - 2) % 4)
    wait_sc((A1_IT - 1) % 4)
    plsc.subcore_barrier()
    pltpu.sync_copy(deg_sh.at[pl.ds(s * slice_sz, slice_sz)], zslice_v)
    pltpu.sync_copy(zslice_v,
                    deg2_hbm.at[pl.ds(c * NPAD + s * slice_sz, slice_sz)])


def _deg_hist(srcf):
    return pl.kernel(
        _a1_body,
        out_type=jax.ShapeDtypeStruct((NC * NPAD,), jnp.float32),
        mesh=_mesh(),
        compiler_params=pltpu.CompilerParams(needs_layout_passes=False),
        scratch_types=(
            [pltpu.VMEM((A1_CH,), jnp.int32)] * 4
            + [pltpu.VMEM((A1_CH,), jnp.float32)]
            + [pltpu.VMEM((NPAD // NS,), jnp.float32)]
            + [pltpu.VMEM_SHARED((NPAD,), jnp.float32)]
            + [pltpu.SemaphoreType.DMA] * 8
        ),
        name="sc_deg_hist",
    )(srcf)


K1_B = 512


def _pack2(a, b):
    ab = lax.bitcast_convert_type(a.astype(jnp.bfloat16),
                                  jnp.uint16).astype(jnp.uint32)
    bb = lax.bitcast_convert_type(b.astype(jnp.bfloat16),
                                  jnp.uint16).astype(jnp.uint32)
    return lax.bitcast_convert_type((ab << 16) | bb, jnp.int32)


def _k1_body(h_ref, fcw_ref, pv_ref, deg2_ref, z_ref, zp_ref):
    h_blk = h_ref[...]
    z = lax.dot_general(h_blk, fcw_ref[...], (((1,), (1,)), ((), ())),
                        preferred_element_type=jnp.float32)
    z_ref[...] = z
    pv = pv_ref[...]
    s4 = lax.dot_general(z, pv[0:4, :], (((1,), (1,)), ((), ())),
                         preferred_element_type=jnp.float32)
    hw = lax.dot_general(h_blk, pv[4:5, :], (((1,), (1,)), ((), ())),
                         preferred_element_type=jnp.float32)
    deg = jnp.maximum(deg2_ref[0, :] + deg2_ref[1, :], 1.0)
    feat = (hw[:, 0] * lax.rsqrt(deg))[:, None]
    zp_ref[...] = jnp.concatenate(
        [_pack2(s4[:, 0:1], s4[:, 1:2]),
         _pack2(s4[:, 2:3], feat),
         lax.bitcast_convert_type(s4[:, 3:4], jnp.int32),
         jnp.zeros((K1_B, 5), jnp.int32)], axis=1)


def _prep(h_pad, fc_w, pvec, deg2):
    return pl.pallas_call(
        _k1_body,
        grid=(NPAD // K1_B,),
        in_specs=[
            pl.BlockSpec((K1_B, D), lambda i: (i, 0)),
            pl.BlockSpec((D, D), lambda i: (0, 0)),
            pl.BlockSpec((8, D), lambda i: (0, 0)),
            pl.BlockSpec((NC, K1_B), lambda i: (0, i)),
        ],
        out_specs=[
            pl.BlockSpec((K1_B, D), lambda i: (i, 0)),
            pl.BlockSpec((K1_B, 8), lambda i: (i, 0)),
        ],
        out_shape=[
            jax.ShapeDtypeStruct((NPAD, D), jnp.float32),
            jax.ShapeDtypeStruct((NPAD, 8), jnp.int32),
        ],
        name="tc_prep",
    )(h_pad, fc_w, pvec, deg2)


def _edge_body(src_hbm, zpf_hbm, z_hbm, smalls_hbm, col_hbm, row_hbm, agg_hbm,
               idx0, idx1, fx0, fx1, sb0, sb1, rw0, rw1, dv0, dv1,
               cv0, cv1, rv0, rv1, ag0, ag1, sm_v,
               semi0, semi1, sems0, sems1, semr0, semr1, semd0, semd1,
               semo0, semo1):
    w = _wid()
    pltpu.sync_copy(smalls_hbm, sm_v)
    idx = [idx0, idx1]
    fx = [fx0, fx1]
    sb = [sb0, sb1]
    rw = [rw0, rw1]
    dv = [dv0, dv1]
    cv = [cv0, cv1]
    rv = [rv0, rv1]
    ag = [ag0, ag1]
    semi = [semi0, semi1]
    sems = [sems0, sems1]
    semr = [semr0, semr1]
    semd = [semd0, semd1]
    semo = [semo0, semo1]

    kio = lax.iota(jnp.int32, 16)
    klt10 = kio < K
    klt9 = kio < (K - 1)
    kcl = jnp.where(klt10, kio, K - 1)
    kcl2 = kcl * 2
    izero = jnp.zeros((16,), jnp.int32)
    esub = lax.shift_right_logical(kio, 1)
    fsub = jnp.bitwise_and(kio, 1)
    m_hi = jnp.full((16,), -65536, jnp.int32)
    wc_vec = sm_v[...]
    wck = [wc_vec[k] for k in range(K)]
    crb = wc_vec[10]
    wbase = w * NODES_W
    n_fx = CH_E * 2

    def issue(c, b):
        n0 = wbase + c * CH_N
        for j in range(n_fx // 16):
            ev = plsc.load_gather(idx[b], [j * 8 + esub])
            fx[b][pl.ds(j * 16, 16)] = ev * 8 + fsub
        for g0, gl in ((0, 128), (128, 128), (256, 64)):
            pltpu.async_copy(zpf_hbm.at[fx[b].at[pl.ds(g0, gl)]],
                             sb[b].at[pl.ds(g0, gl)], sems[b])
        for r0 in range(0, CH_E, 40):
            pltpu.async_copy(z_hbm.at[idx[b].at[pl.ds(r0, 40)]],
                             rw[b].at[pl.ds(r0, 40)], semr[b])
        pltpu.async_copy(zpf_hbm.at[pl.ds(n0 * 8, CH_N * 8)], dv[b], semd[b])

    def prefetch_idx(c, b):
        pltpu.async_copy(src_hbm.at[pl.ds((wbase + c * CH_N) * K, CH_E)],
                         idx[b], semi[b])

    def wait_idx(b):
        pltpu.make_async_copy(src_hbm.at[pl.ds(0, CH_E)], idx[b],
                              semi[b]).wait()

    def wait_data(b):
        pltpu.make_async_copy(zpf_hbm.at[pl.ds(0, n_fx)], sb[b],
                              sems[b]).wait()
        pltpu.make_async_copy(z_hbm.at[pl.ds(0, CH_E)], rw[b],
                              semr[b]).wait()
        pltpu.make_async_copy(zpf_hbm.at[pl.ds(0, CH_N * 8)], dv[b],
                              semd[b]).wait()

    def wait_out(b):
        pltpu.make_async_copy(col_hbm.at[pl.ds(0, CH_N)], cv[b],
                              semo[b]).wait()
        pltpu.make_async_copy(row_hbm.at[pl.ds(0, CH_N * 16)], rv[b],
                              semo[b]).wait()
        pltpu.make_async_copy(agg_hbm.at[pl.ds(0, CH_N)], ag[b],
                              semo[b]).wait()

    def compute(c, b):
        agg_reg = jnp.zeros((16,), jnp.float32)
        for n in range(CH_N):
            base = n * K * 2
            w0v = plsc.load_gather(sb[b], [base + kcl2])
            w1v = plsc.load_gather(sb[b], [base + kcl2 + 1])
            al = plsc.bitcast(jnp.bitwise_and(w0v, m_hi), jnp.float32)
            q0 = plsc.bitcast(lax.shift_left(w0v, 16), jnp.float32)
            q1 = plsc.bitcast(jnp.bitwise_and(w1v, m_hi), jnp.float32)
            ft = plsc.bitcast(lax.shift_left(w1v, 16), jnp.float32)
            zr = plsc.bitcast(plsc.load_gather(dv[b], [izero + (n * 8 + 2)]),
                              jnp.float32)
            e = al + zr
            e = jnp.where(e >= 0.0, e, 0.01 * e)
            em = jnp.where(klt10, e, -3.0e38)
            m = jnp.max(em)
            ex = jnp.where(klt10, jnp.exp(e - m), 0.0)
            alpha = ex / jnp.sum(ex)
            b1 = alpha * q1
            rv[b][pl.ds(n * 16, 16)] = b1
            b1s = plsc.load_gather(rv[b], [n * 16 + jnp.minimum(kio + 1, 15)])
            rr = alpha * q0 + b1s + crb
            rv[b][pl.ds(n * 16, 16)] = jnp.where(klt9, rr, 0.0)
            agg_reg = jnp.where(kio == n,
                                jnp.sum(jnp.where(klt10, ft, 0.0)), agg_reg)
            acc = [jnp.zeros((16,), jnp.float32) for _ in range(D // 16)]
            for k in range(K):
                bk = alpha[k] * wck[k]
                for dd in range(D // 16):
                    acc[dd] = acc[dd] + bk * rw[b][n * K + k,
                                                   pl.ds(dd * 16, 16)]
            for dd in range(D // 16):
                cv[b][n, pl.ds(dd * 16, 16)] = acc[dd]
        ag[b][...] = agg_reg
        n0 = wbase + c * CH_N
        pltpu.async_copy(cv[b], col_hbm.at[pl.ds(n0, CH_N)], semo[b])
        pltpu.async_copy(rv[b], row_hbm.at[pl.ds(n0 * 16, CH_N * 16)],
                         semo[b])
        pltpu.async_copy(ag[b], agg_hbm.at[pl.ds(n0, CH_N)], semo[b])

    pltpu.sync_copy(src_hbm.at[pl.ds(wbase * K, CH_E)], idx0)
    issue(0, 0)
    prefetch_idx(1, 1)

    def body(i2, _):
        for b in (0, 1):
            c = i2 * 2 + b
            nb = 1 - b
            nc = c + 1

            @pl.when(nc < CH_IT)
            def _():
                wait_idx(nb)
                issue(nc, nb)

            wait_data(b)

            @pl.when(nc + 1 < CH_IT)
            def _():
                prefetch_idx(nc + 1, b)

            @pl.when(c >= 2)
            def _():
                wait_out(b)

            compute(c, b)
        return 0

    lax.fori_loop(0, CH_IT // 2, body, 0)
    wait_out(0)
    wait_out(1)


def _edge_all(srcf, zpf, z, smalls):
    return pl.kernel(
        _edge_body,
        out_type=[
            jax.ShapeDtypeStruct((NPAD, D), jnp.float32),
            jax.ShapeDtypeStruct((NPAD * 16,), jnp.float32),
            jax.ShapeDtypeStruct((NPAD,), jnp.float32),
        ],
        mesh=_mesh(),
        compiler_params=pltpu.CompilerParams(needs_layout_passes=False),
        scratch_types=(
            [pltpu.VMEM((CH_E,), jnp.int32)] * 2
            + [pltpu.VMEM((CH_E * 2,), jnp.int32)] * 2
            + [pltpu.VMEM((CH_E * 2,), jnp.int32)] * 2
            + [pltpu.VMEM((CH_E, D), jnp.float32)] * 2
            + [pltpu.VMEM((CH_N * 8,), jnp.int32)] * 2
            + [pltpu.VMEM((CH_N, D), jnp.float32)] * 2
            + [pltpu.VMEM((CH_N * 16,), jnp.float32)] * 2
            + [pltpu.VMEM((CH_N,), jnp.float32)] * 2
            + [pltpu.VMEM((16,), jnp.float32)]
            + [pltpu.SemaphoreType.DMA] * 10
        ),
        name="sc_edge_all",
    )(srcf, zpf, z, smalls)


def _c2_body(agg_ref, sv_ref, w_ref):
    a = agg_ref[...]
    r = lax.broadcasted_iota(jnp.int32, a.shape, 0)
    l = lax.broadcasted_iota(jnp.int32, a.shape, 1)
    nidx = r * 128 + l
    valid = nidx < N
    s = a * (float(K) ** -0.5) + sv_ref[0, 0:1]
    sm = jnp.where(valid, s, -3.0e38)
    m = jnp.max(sm)
    ex = jnp.where(valid, jnp.exp(s - m), 0.0)
    w_ref[...] = ex / (jnp.sum(ex) * float(N))


def _node_weights(agg2d, svec):
    return pl.pallas_call(
        _c2_body,
        grid=(1,),
        in_specs=[
            pl.BlockSpec((NPAD // 128, 128), lambda i: (0, 0)),
            pl.BlockSpec((1, 128), lambda i: (0, 0)),
        ],
        out_specs=pl.BlockSpec((NPAD // 128, 128), lambda i: (0, 0)),
        out_shape=jax.ShapeDtypeStruct((NPAD // 128, 128), jnp.float32),
        name="tc_weights",
    )(agg2d, svec)


E_B = 1024


def _e1_body(col_ref, row_ref, st_ref):
    i = pl.program_id(0)
    c = col_ref[...]
    s1c = jnp.sum(c)
    s2c = jnp.sum(c * c)
    rw = row_ref[...]
    r = lax.broadcasted_iota(jnp.int32, rw.shape, 0)
    rw = jnp.where(i * E_B + r < N, rw, 0.0)
    s1r = jnp.sum(rw)
    s2r = jnp.sum(rw * rw)
    lane = lax.broadcasted_iota(jnp.int32, (1, 128), 1)
    contrib = (jnp.where(lane == 0, s1c, 0.0)
               + jnp.where(lane == 1, s2c, 0.0)
               + jnp.where(lane == 2, s1r, 0.0)
               + jnp.where(lane == 3, s2r, 0.0))

    @pl.when(i == 0)
    def _():
        st_ref[...] = jnp.zeros((1, 128), jnp.float32)

    st_ref[...] += contrib


def _bn_stats(col, row2d):
    return pl.pallas_call(
        _e1_body,
        grid=(NPAD // E_B,),
        in_specs=[
            pl.BlockSpec((E_B, D), lambda i: (i, 0)),
            pl.BlockSpec((E_B, 16), lambda i: (i, 0)),
        ],
        out_specs=pl.BlockSpec((1, 128), lambda i: (0, 0)),
        out_shape=jax.ShapeDtypeStruct((1, 128), jnp.float32),
        name="tc_bn_stats",
    )(col, row2d)


def _e2_body(col_ref, row_ref, w_ref, h_ref, st_ref, bn_ref, l1_ref, l2_ref,
             hp_ref, cw_ref, cb_ref, out_ref, acc_ref):
    i = pl.program_id(0)
    st = st_ref[0, :]
    s1c, s2c = st[0:1], st[1:2]
    s1r, s2r = st[2:3], st[3:4]
    muc = s1c / float(N * D)
    varc = s2c / float(N * D) - muc * muc
    mur = s1r / float(N * (K - 1))
    varr = s2r / float(N * (K - 1)) - mur * mur
    gr, br = bn_ref[0, 0:1], bn_ref[1, 0:1]
    gc, bc = bn_ref[2, 0:1], bn_ref[3, 0:1]
    ac = gc * lax.rsqrt(varc + 1e-5)
    bcs = bc - muc * ac
    ar = gr * lax.rsqrt(varr + 1e-5)
    brs = br - mur * ar

    coln = jnp.maximum(col_ref[...] * ac + bcs, 0.0)
    rown = jnp.maximum(row_ref[...] * ar + brs, 0.0)
    uf = (lax.dot_general(rown, l1_ref[...], (((1,), (0,)), ((), ())),
                          preferred_element_type=jnp.float32)
          + lax.dot_general(coln, l2_ref[...], (((1,), (0,)), ((), ())),
                            preferred_element_type=jnp.float32)
          + lax.dot_general(h_ref[...], hp_ref[...], (((1,), (1,)), ((), ())),
                            preferred_element_type=jnp.float32))
    uf = jnp.maximum(uf, 0.0)
    part = jnp.sum(w_ref[...] * uf, axis=0, keepdims=True)

    @pl.when(i == 0)
    def _():
        acc_ref[...] = jnp.zeros((1, 128), jnp.float32)

    acc_ref[...] += part

    @pl.when(i == (NPAD // E_B) - 1)
    def _():
        out_ref[...] = lax.dot_general(
            acc_ref[...], cw_ref[...], (((1,), (1,)), ((), ())),
            preferred_element_type=jnp.float32) + cb_ref[...]


def _final(col, row2d, w1d, h_pad, stats, bnvec, l1p, l2, hpw, cw, cb):
    return pl.pallas_call(
        _e2_body,
        grid=(NPAD // E_B,),
        in_specs=[
            pl.BlockSpec((E_B, D), lambda i: (i, 0)),
            pl.BlockSpec((E_B, 16), lambda i: (i, 0)),
            pl.BlockSpec((E_B, 1), lambda i: (i, 0)),
            pl.BlockSpec((E_B, D), lambda i: (i, 0)),
            pl.BlockSpec((1, 128), lambda i: (0, 0)),
            pl.BlockSpec((8, 128), lambda i: (0, 0)),
            pl.BlockSpec((16, D), lambda i: (0, 0)),
            pl.BlockSpec((D, D), lambda i: (0, 0)),
            pl.BlockSpec((D, D), lambda i: (0, 0)),
            pl.BlockSpec((NCLS, D), lambda i: (0, 0)),
            pl.BlockSpec((1, NCLS), lambda i: (0, 0)),
        ],
        out_specs=pl.BlockSpec((1, NCLS), lambda i: (0, 0)),
        out_shape=jax.ShapeDtypeStruct((1, NCLS), jnp.float32),
        scratch_shapes=[pltpu.VMEM((1, 128), jnp.float32)],
        name="tc_final",
    )(col, row2d, w1d, h_pad, stats, bnvec, l1p, l2, hpw, cw, cb)


def kernel(h, src_idx, fc_w, attn_w, convrow_w, convrow_b, bn_row_g, bn_row_b,
           convcol_w, convcol_b, bn_col_g, bn_col_b, localw, h_proj_w,
           conv_w, conv_b, classify_w, classify_b):
    f32 = jnp.float32
    h_pad = jnp.concatenate([h, jnp.zeros((NPAD - N, D), f32)], axis=0)
    srcf = jnp.concatenate(
        [src_idx,
         jnp.full((NPAD - N, K), NPAD - 1, jnp.int32)], axis=0).reshape(-1)

    a_l = attn_w[0, :D]
    a_r = attn_w[0, D:]
    w0 = convrow_w[0, 0, 0, :]
    w1 = convrow_w[0, 0, 1, :]
    wc = convcol_w[0, 0, :, 0]
    cwv = conv_w[:, 0]
    pvec = jnp.concatenate(
        [jnp.stack([a_l, w0, w1, a_r, cwv], axis=0),
         jnp.zeros((3, D), f32)], axis=0)
    smalls = jnp.concatenate(
        [wc, convrow_b, jnp.zeros((5,), f32)], axis=0)
    svec = jnp.broadcast_to(conv_b[0], (1, 128)).astype(f32)
    bnvec = jnp.stack([
        jnp.broadcast_to(bn_row_g[0], (128,)),
        jnp.broadcast_to(bn_row_b[0], (128,)),
        jnp.broadcast_to(bn_col_g[0], (128,)),
        jnp.broadcast_to(bn_col_b[0], (128,)),
    ] + [jnp.zeros((128,), f32)] * 4, axis=0)
    l1p = jnp.concatenate(
        [localw[:K - 1, :], jnp.zeros((16 - (K - 1), D), f32)], axis=0)
    l2 = localw[K - 1:, :]
    cb = classify_b.reshape(1, NCLS)

    deg2 = _deg_hist(srcf).reshape(NC, NPAD)
    z, zp = _prep(h_pad, fc_w, pvec, deg2)
    col, row_f, agg = _edge_all(srcf, zp.reshape(-1), z, smalls)
    w1d = _node_weights(agg.reshape(NPAD // 128, 128), svec).reshape(NPAD, 1)
    row2d = row_f.reshape(NPAD, 16)
    stats = _bn_stats(col, row2d)
    return _final(col, row2d, w1d, h_pad, stats, bnvec, l1p, l2,
                  h_proj_w, classify_w, cb)


# speedup vs baseline: 1.0223x; 1.0223x over previous
import functools

import jax
import jax.numpy as jnp
from jax import lax
from jax.experimental import pallas as pl
from jax.experimental.pallas import tpu as pltpu
from jax.experimental.pallas import tpu_sc as plsc

N = 50000
D = 128
K = 10
NCLS = 16

NC = 2
NS = 16
NW = NC * NS
NPAD = 50176
NODES_W = NPAD // NW
EDGES_W = NODES_W * K

A1_CH = 112
A1_IT = EDGES_W // A1_CH

CH_N = 16
CH_E = CH_N * K
CH_IT = NODES_W // CH_N

_mesh = functools.partial(plsc.VectorSubcoreMesh,
                          core_axis_name="c", subcore_axis_name="s")


def _wid():
    return lax.axis_index("c") * NS + lax.axis_index("s")


def _a1_body(src_hbm, deg2_hbm, i0, i1, i2, i3, ones_v, zslice_v, deg_sh,
             si0, si1, si2, si3, ss0, ss1, ss2, ss3):
    c = lax.axis_index("c")
    s = lax.axis_index("s")
    w = _wid()
    idx = [i0, i1, i2, i3]
    semi = [si0, si1, si2, si3]
    sems = [ss0, ss1, ss2, ss3]
    zero16 = jnp.zeros((16,), jnp.float32)
    for j in range(A1_CH // 16):
        ones_v[pl.ds(j * 16, 16)] = zero16 + 1.0

    slice_sz = NPAD // NS

    def zbody(i, _):
        zslice_v[pl.ds(i * 16, 16)] = zero16
        return 0

    lax.fori_loop(0, slice_sz // 16, zbody, 0)
    pltpu.sync_copy(zslice_v, deg_sh.at[pl.ds(s * slice_sz, slice_sz)])
    plsc.subcore_barrier()

    def pf(cc, u):
        pltpu.async_copy(src_hbm.at[pl.ds(w * EDGES_W + cc * A1_CH, A1_CH)],
                         idx[u], semi[u])

    def wait_idx(u):
        pltpu.make_async_copy(src_hbm.at[pl.ds(0, A1_CH)], idx[u],
                              semi[u]).wait()

    def wait_sc(u):
        pltpu.make_async_copy(src_hbm.at[pl.ds(0, A1_CH)], ones_v,
                              sems[u]).wait()

    pf(0, 0)
    pf(1, 1)

    def body(c4, _):
        for u in range(4):
            cc = c4 * 4 + u

            @pl.when(cc >= 2)
            def _():
                wait_sc((u + 2) % 4)

            @pl.when(cc + 2 < A1_IT)
            def _():
                pf(cc + 2, (u + 2) % 4)

            wait_idx(u)
            pltpu.async_copy(ones_v, deg_sh.at[idx[u]], sems[u], add=True)
        return 0

    lax.fori_loop(0, A1_IT // 4, body, 0)
    wait_sc((A1_IT - 2) % 4)
    wait_sc((A1_IT - 1) % 4)
    plsc.subcore_barrier()
    pltpu.sync_copy(deg_sh.at[pl.ds(s * slice_sz, slice_sz)], zslice_v)
    pltpu.sync_copy(zslice_v,
                    deg2_hbm.at[pl.ds(c * NPAD + s * slice_sz, slice_sz)])


def _deg_hist(srcf):
    return pl.kernel(
        _a1_body,
        out_type=jax.ShapeDtypeStruct((NC * NPAD,), jnp.float32),
        mesh=_mesh(),
        compiler_params=pltpu.CompilerParams(needs_layout_passes=False),
        scratch_types=(
            [pltpu.VMEM((A1_CH,), jnp.int32)] * 4
            + [pltpu.VMEM((A1_CH,), jnp.float32)]
            + [pltpu.VMEM((NPAD // NS,), jnp.float32)]
            + [pltpu.VMEM_SHARED((NPAD,), jnp.float32)]
            + [pltpu.SemaphoreType.DMA] * 8
        ),
        name="sc_deg_hist",
    )(srcf)


K1_B = 512


def _pack2(a, b):
    ab = lax.bitcast_convert_type(a.astype(jnp.bfloat16),
                                  jnp.uint16).astype(jnp.uint32)
    bb = lax.bitcast_convert_type(b.astype(jnp.bfloat16),
                                  jnp.uint16).astype(jnp.uint32)
    return lax.bitcast_convert_type((ab << 16) | bb, jnp.int32)


def _k1_body(h_ref, fcw_ref, pv_ref, deg2_ref, z_ref, zp_ref):
    h_blk = h_ref[...]
    z = lax.dot_general(h_blk, fcw_ref[...], (((1,), (1,)), ((), ())),
                        preferred_element_type=jnp.float32)
    z_ref[...] = z
    pv = pv_ref[...]
    s4 = lax.dot_general(z, pv[0:4, :], (((1,), (1,)), ((), ())),
                         preferred_element_type=jnp.float32)
    hw = lax.dot_general(h_blk, pv[4:5, :], (((1,), (1,)), ((), ())),
                         preferred_element_type=jnp.float32)
    deg = jnp.maximum(deg2_ref[0, :] + deg2_ref[1, :], 1.0)
    feat = (hw[:, 0] * lax.rsqrt(deg))[:, None]
    zp_ref[...] = jnp.concatenate(
        [_pack2(s4[:, 0:1], s4[:, 1:2]),
         _pack2(s4[:, 2:3], feat),
         lax.bitcast_convert_type(s4[:, 3:4], jnp.int32),
         jnp.zeros((K1_B, 5), jnp.int32)], axis=1)


def _prep(h_pad, fc_w, pvec, deg2):
    return pl.pallas_call(
        _k1_body,
        grid=(NPAD // K1_B,),
        in_specs=[
            pl.BlockSpec((K1_B, D), lambda i: (i, 0)),
            pl.BlockSpec((D, D), lambda i: (0, 0)),
            pl.BlockSpec((8, D), lambda i: (0, 0)),
            pl.BlockSpec((NC, K1_B), lambda i: (0, i)),
        ],
        out_specs=[
            pl.BlockSpec((K1_B, D), lambda i: (i, 0)),
            pl.BlockSpec((K1_B, 8), lambda i: (i, 0)),
        ],
        out_shape=[
            jax.ShapeDtypeStruct((NPAD, D), jnp.float32),
            jax.ShapeDtypeStruct((NPAD, 8), jnp.int32),
        ],
        name="tc_prep",
    )(h_pad, fc_w, pvec, deg2)


def _edge_body(src_hbm, zpf_hbm, z_hbm, smalls_hbm, col_hbm, row_hbm, agg_hbm,
               idx0, idx1, fx0, fx1, sb0, sb1, rw0, rw1, dv0, dv1,
               cv0, cv1, rv0, rv1, ag0, ag1, sm_v,
               semi0, semi1, sems0, sems1, semr0, semr1, semd0, semd1,
               semo0, semo1):
    w = _wid()
    pltpu.sync_copy(smalls_hbm, sm_v)
    idx = [idx0, idx1]
    fx = [fx0, fx1]
    sb = [sb0, sb1]
    rw = [rw0, rw1]
    dv = [dv0, dv1]
    cv = [cv0, cv1]
    rv = [rv0, rv1]
    ag = [ag0, ag1]
    semi = [semi0, semi1]
    sems = [sems0, sems1]
    semr = [semr0, semr1]
    semd = [semd0, semd1]
    semo = [semo0, semo1]

    kio = lax.iota(jnp.int32, 16)
    klt10 = kio < K
    klt9 = kio < (K - 1)
    kcl = jnp.where(klt10, kio, K - 1)
    kcl2 = kcl * 2
    izero = jnp.zeros((16,), jnp.int32)
    esub = lax.shift_right_logical(kio, 1)
    fsub = jnp.bitwise_and(kio, 1)
    m_hi = jnp.full((16,), -65536, jnp.int32)
    wc_vec = sm_v[...]
    wck = [wc_vec[k] for k in range(K)]
    crb = wc_vec[10]
    wbase = w * NODES_W
    n_fx = CH_E * 2

    def issue(c, b):
        n0 = wbase + c * CH_N
        for j in range(n_fx // 16):
            ev = plsc.load_gather(idx[b], [j * 8 + esub])
            fx[b][pl.ds(j * 16, 16)] = ev * 8 + fsub
        for g0, gl in ((0, 128), (128, 128), (256, 64)):
            pltpu.async_copy(zpf_hbm.at[fx[b].at[pl.ds(g0, gl)]],
                             sb[b].at[pl.ds(g0, gl)], sems[b])
        for r0 in range(0, CH_E, 40):
            pltpu.async_copy(z_hbm.at[idx[b].at[pl.ds(r0, 40)]],
                             rw[b].at[pl.ds(r0, 40)], semr[b])
        pltpu.async_copy(zpf_hbm.at[pl.ds(n0 * 8, CH_N * 8)], dv[b], semd[b])

    def prefetch_idx(c, b):
        pltpu.async_copy(src_hbm.at[pl.ds((wbase + c * CH_N) * K, CH_E)],
                         idx[b], semi[b])

    def wait_idx(b):
        pltpu.make_async_copy(src_hbm.at[pl.ds(0, CH_E)], idx[b],
                              semi[b]).wait()

    def wait_data(b):
        pltpu.make_async_copy(zpf_hbm.at[pl.ds(0, n_fx)], sb[b],
                              sems[b]).wait()
        pltpu.make_async_copy(z_hbm.at[pl.ds(0, CH_E)], rw[b],
                              semr[b]).wait()
        pltpu.make_async_copy(zpf_hbm.at[pl.ds(0, CH_N * 8)], dv[b],
                              semd[b]).wait()

    def wait_out(b):
        pltpu.make_async_copy(col_hbm.at[pl.ds(0, CH_N)], cv[b],
                              semo[b]).wait()
        pltpu.make_async_copy(row_hbm.at[pl.ds(0, CH_N * 16)], rv[b],
                              semo[b]).wait()
        pltpu.make_async_copy(agg_hbm.at[pl.ds(0, CH_N)], ag[b],
                              semo[b]).wait()

    def compute(c, b):
        agg_reg = jnp.zeros((16,), jnp.float32)
        for n in range(CH_N):
            base = n * K * 2
            w0v = plsc.load_gather(sb[b], [base + kcl2])
            w1v = plsc.load_gather(sb[b], [base + kcl2 + 1])
            al = plsc.bitcast(jnp.bitwise_and(w0v, m_hi), jnp.float32)
            q0 = plsc.bitcast(lax.shift_left(w0v, 16), jnp.float32)
            q1 = plsc.bitcast(jnp.bitwise_and(w1v, m_hi), jnp.float32)
            ft = plsc.bitcast(lax.shift_left(w1v, 16), jnp.float32)
            zr = plsc.bitcast(plsc.load_gather(dv[b], [izero + (n * 8 + 2)]),
                              jnp.float32)
            e = al + zr
            e = jnp.where(e >= 0.0, e, 0.01 * e)
            em = jnp.where(klt10, e, -3.0e38)
            m = jnp.max(em)
            ex = jnp.where(klt10, jnp.exp(e - m), 0.0)
            alpha = ex / jnp.sum(ex)
            b1 = alpha * q1
            rv[b][pl.ds(n * 16, 16)] = b1
            b1s = plsc.load_gather(rv[b], [n * 16 + jnp.minimum(kio + 1, 15)])
            rr = alpha * q0 + b1s + crb
            rv[b][pl.ds(n * 16, 16)] = jnp.where(klt9, rr, 0.0)
            agg_reg = jnp.where(kio == n,
                                jnp.sum(jnp.where(klt10, ft, 0.0)), agg_reg)
            acc = [jnp.zeros((16,), jnp.float32) for _ in range(D // 16)]
            for k in range(K):
                bk = alpha[k] * wck[k]
                for dd in range(D // 16):
                    acc[dd] = acc[dd] + bk * rw[b][n * K + k,
                                                   pl.ds(dd * 16, 16)]
            for dd in range(D // 16):
                cv[b][n, pl.ds(dd * 16, 16)] = acc[dd]
        ag[b][...] = agg_reg
        n0 = wbase + c * CH_N
        pltpu.async_copy(cv[b], col_hbm.at[pl.ds(n0, CH_N)], semo[b])
        pltpu.async_copy(rv[b], row_hbm.at[pl.ds(n0 * 16, CH_N * 16)],
                         semo[b])
        pltpu.async_copy(ag[b], agg_hbm.at[pl.ds(n0, CH_N)], semo[b])

    pltpu.sync_copy(src_hbm.at[pl.ds(wbase * K, CH_E)], idx0)
    issue(0, 0)
    prefetch_idx(1, 1)

    def body(i2, _):
        for b in (0, 1):
            c = i2 * 2 + b
            nb = 1 - b
            nc = c + 1

            @pl.when(nc < CH_IT)
            def _():
                wait_idx(nb)
                issue(nc, nb)

            wait_data(b)

            @pl.when(nc + 1 < CH_IT)
            def _():
                prefetch_idx(nc + 1, b)

            @pl.when(c >= 2)
            def _():
                wait_out(b)

            compute(c, b)
        return 0

    lax.fori_loop(0, CH_IT // 2, body, 0)
    wait_out(0)
    wait_out(1)


def _edge_all(srcf, zpf, z, smalls):
    return pl.kernel(
        _edge_body,
        out_type=[
            jax.ShapeDtypeStruct((NPAD, D), jnp.float32),
            jax.ShapeDtypeStruct((NPAD * 16,), jnp.float32),
            jax.ShapeDtypeStruct((NPAD,), jnp.float32),
        ],
        mesh=_mesh(),
        compiler_params=pltpu.CompilerParams(needs_layout_passes=False),
        scratch_types=(
            [pltpu.VMEM((CH_E,), jnp.int32)] * 2
            + [pltpu.VMEM((CH_E * 2,), jnp.int32)] * 2
            + [pltpu.VMEM((CH_E * 2,), jnp.int32)] * 2
            + [pltpu.VMEM((CH_E, D), jnp.float32)] * 2
            + [pltpu.VMEM((CH_N * 8,), jnp.int32)] * 2
            + [pltpu.VMEM((CH_N, D), jnp.float32)] * 2
            + [pltpu.VMEM((CH_N * 16,), jnp.float32)] * 2
            + [pltpu.VMEM((CH_N,), jnp.float32)] * 2
            + [pltpu.VMEM((16,), jnp.float32)]
            + [pltpu.SemaphoreType.DMA] * 10
        ),
        name="sc_edge_all",
    )(srcf, zpf, z, smalls)


E_B = 1024


def _e1_body(col_ref, row_ref, agg_ref, st_ref):
    i = pl.program_id(0)
    c = col_ref[...]
    s1c = jnp.sum(c)
    s2c = jnp.sum(c * c)
    rw = row_ref[...]
    r = lax.broadcasted_iota(jnp.int32, rw.shape, 0)
    rw = jnp.where(i * E_B + r < N, rw, 0.0)
    s1r = jnp.sum(rw)
    s2r = jnp.sum(rw * rw)
    lane = lax.broadcasted_iota(jnp.int32, (1, 128), 1)
    contrib = (jnp.where(lane == 0, s1c, 0.0)
               + jnp.where(lane == 1, s2c, 0.0)
               + jnp.where(lane == 2, s1r, 0.0)
               + jnp.where(lane == 3, s2r, 0.0))

    @pl.when(i == 0)
    def _():
        a = agg_ref[...]
        rr = lax.broadcasted_iota(jnp.int32, a.shape, 0)
        ll = lax.broadcasted_iota(jnp.int32, a.shape, 1)
        valid = rr * 128 + ll < N
        s = a * (float(K) ** -0.5)
        m = jnp.max(jnp.where(valid, s, -3.0e38))
        zsum = jnp.sum(jnp.where(valid, jnp.exp(s - m), 0.0))
        st_ref[...] = (jnp.where(lane == 4, m, 0.0)
                       + jnp.where(lane == 5, zsum, 0.0))

    st_ref[...] += contrib


def _bn_stats(col, row2d, aggd):
    return pl.pallas_call(
        _e1_body,
        grid=(NPAD // E_B,),
        in_specs=[
            pl.BlockSpec((E_B, D), lambda i: (i, 0)),
            pl.BlockSpec((E_B, 16), lambda i: (i, 0)),
            pl.BlockSpec((NPAD // 128, 128), lambda i: (0, 0)),
        ],
        out_specs=pl.BlockSpec((1, 128), lambda i: (0, 0)),
        out_shape=jax.ShapeDtypeStruct((1, 128), jnp.float32),
        name="tc_bn_stats",
    )(col, row2d, aggd)


def _e2_body(col_ref, row_ref, agg_ref, h_ref, st_ref, bn_ref, l1_ref, l2_ref,
             hp_ref, cw_ref, cb_ref, out_ref, acc_ref):
    i = pl.program_id(0)
    st = st_ref[0, :]
    s1c, s2c = st[0:1], st[1:2]
    s1r, s2r = st[2:3], st[3:4]
    m, zsum = st[4:5], st[5:6]
    muc = s1c / float(N * D)
    varc = s2c / float(N * D) - muc * muc
    mur = s1r / float(N * (K - 1))
    varr = s2r / float(N * (K - 1)) - mur * mur
    gr, br = bn_ref[0, 0:1], bn_ref[1, 0:1]
    gc, bc = bn_ref[2, 0:1], bn_ref[3, 0:1]
    ac = gc * lax.rsqrt(varc + 1e-5)
    bcs = bc - muc * ac
    ar = gr * lax.rsqrt(varr + 1e-5)
    brs = br - mur * ar

    coln = jnp.maximum(col_ref[...] * ac + bcs, 0.0)
    rown = jnp.maximum(row_ref[...] * ar + brs, 0.0)
    uf = (lax.dot_general(rown, l1_ref[...], (((1,), (0,)), ((), ())),
                          preferred_element_type=jnp.float32)
          + lax.dot_general(coln, l2_ref[...], (((1,), (0,)), ((), ())),
                            preferred_element_type=jnp.float32)
          + lax.dot_general(h_ref[...], hp_ref[...], (((1,), (1,)), ((), ())),
                            preferred_element_type=jnp.float32))
    uf = jnp.maximum(uf, 0.0)
    a = agg_ref[...]
    rr = lax.broadcasted_iota(jnp.int32, a.shape, 0)
    ll = lax.broadcasted_iota(jnp.int32, a.shape, 1)
    valid = (i * 8 + rr) * 128 + ll < N
    s = a * (float(K) ** -0.5)
    wblk = jnp.where(valid, jnp.exp(s - m), 0.0) / (zsum * float(N))
    part = jnp.zeros((1, 128), jnp.float32)
    for r in range(8):
        part = part + lax.dot_general(
            wblk[r:r + 1, :], uf[r * 128:(r + 1) * 128, :],
            (((1,), (0,)), ((), ())), preferred_element_type=jnp.float32)

    @pl.when(i == 0)
    def _():
        acc_ref[...] = jnp.zeros((1, 128), jnp.float32)

    acc_ref[...] += part

    @pl.when(i == (NPAD // E_B) - 1)
    def _():
        out_ref[...] = lax.dot_general(
            acc_ref[...], cw_ref[...], (((1,), (1,)), ((), ())),
            preferred_element_type=jnp.float32) + cb_ref[...]


def _final(col, row2d, aggd, h_pad, stats, bnvec, l1p, l2, hpw, cw, cb):
    return pl.pallas_call(
        _e2_body,
        grid=(NPAD // E_B,),
        in_specs=[
            pl.BlockSpec((E_B, D), lambda i: (i, 0)),
            pl.BlockSpec((E_B, 16), lambda i: (i, 0)),
            pl.BlockSpec((8, 128), lambda i: (i, 0)),
            pl.BlockSpec((E_B, D), lambda i: (i, 0)),
            pl.BlockSpec((1, 128), lambda i: (0, 0)),
            pl.BlockSpec((8, 128), lambda i: (0, 0)),
            pl.BlockSpec((16, D), lambda i: (0, 0)),
            pl.BlockSpec((D, D), lambda i: (0, 0)),
            pl.BlockSpec((D, D), lambda i: (0, 0)),
            pl.BlockSpec((NCLS, D), lambda i: (0, 0)),
            pl.BlockSpec((1, NCLS), lambda i: (0, 0)),
        ],
        out_specs=pl.BlockSpec((1, NCLS), lambda i: (0, 0)),
        out_shape=jax.ShapeDtypeStruct((1, NCLS), jnp.float32),
        scratch_shapes=[pltpu.VMEM((1, 128), jnp.float32)],
        name="tc_final",
    )(col, row2d, aggd, h_pad, stats, bnvec, l1p, l2, hpw, cw, cb)


def kernel(h, src_idx, fc_w, attn_w, convrow_w, convrow_b, bn_row_g, bn_row_b,
           convcol_w, convcol_b, bn_col_g, bn_col_b, localw, h_proj_w,
           conv_w, conv_b, classify_w, classify_b):
    f32 = jnp.float32
    h_pad = jnp.concatenate([h, jnp.zeros((NPAD - N, D), f32)], axis=0)
    srcf = jnp.concatenate(
        [src_idx,
         jnp.full((NPAD - N, K), NPAD - 1, jnp.int32)], axis=0).reshape(-1)

    a_l = attn_w[0, :D]
    a_r = attn_w[0, D:]
    w0 = convrow_w[0, 0, 0, :]
    w1 = convrow_w[0, 0, 1, :]
    wc = convcol_w[0, 0, :, 0]
    cwv = conv_w[:, 0]
    pvec = jnp.concatenate(
        [jnp.stack([a_l, w0, w1, a_r, cwv], axis=0),
         jnp.zeros((3, D), f32)], axis=0)
    smalls = jnp.concatenate(
        [wc, convrow_b, jnp.zeros((5,), f32)], axis=0)
    bnvec = jnp.stack([
        jnp.broadcast_to(bn_row_g[0], (128,)),
        jnp.broadcast_to(bn_row_b[0], (128,)),
        jnp.broadcast_to(bn_col_g[0], (128,)),
        jnp.broadcast_to(bn_col_b[0], (128,)),
    ] + [jnp.zeros((128,), f32)] * 4, axis=0)
    l1p = jnp.concatenate(
        [localw[:K - 1, :], jnp.zeros((16 - (K - 1), D), f32)], axis=0)
    l2 = localw[K - 1:, :]
    cb = classify_b.reshape(1, NCLS)

    deg2 = _deg_hist(srcf).reshape(NC, NPAD)
    z, zp = _prep(h_pad, fc_w, pvec, deg2)
    col, row_f, agg = _edge_all(srcf, zp.reshape(-1), z, smalls)
    aggd = agg.reshape(NPAD // 128, 128)
    row2d = row_f.reshape(NPAD, 16)
    stats = _bn_stats(col, row2d, aggd)
    return _final(col, row2d, aggd, h_pad, stats, bnvec, l1p, l2,
                  h_proj_w, classify_w, cb)


# speedup vs baseline: 1.0626x; 1.0394x over previous
import functools

import jax
import jax.numpy as jnp
from jax import lax
from jax.experimental import pallas as pl
from jax.experimental.pallas import tpu as pltpu
from jax.experimental.pallas import tpu_sc as plsc

N = 50000
D = 128
K = 10
NCLS = 16

NC = 2
NS = 16
NW = NC * NS
NPAD = 50176
NODES_W = NPAD // NW
EDGES_W = NODES_W * K

A1_CH = 112
A1_IT = EDGES_W // A1_CH

CH_N = 16
CH_E = CH_N * K
CH_IT = NODES_W // CH_N

_mesh = functools.partial(plsc.VectorSubcoreMesh,
                          core_axis_name="c", subcore_axis_name="s")


def _wid():
    return lax.axis_index("c") * NS + lax.axis_index("s")


def _a1_body(src_hbm, deg2_hbm, i0, i1, i2, i3, ones_v, zslice_v, deg_sh,
             si0, si1, si2, si3, ss0, ss1, ss2, ss3):
    c = lax.axis_index("c")
    s = lax.axis_index("s")
    w = _wid()
    idx = [i0, i1, i2, i3]
    semi = [si0, si1, si2, si3]
    sems = [ss0, ss1, ss2, ss3]
    zero16 = jnp.zeros((16,), jnp.float32)
    for j in range(A1_CH // 16):
        ones_v[pl.ds(j * 16, 16)] = zero16 + 1.0

    slice_sz = NPAD // NS

    def zbody(i, _):
        zslice_v[pl.ds(i * 16, 16)] = zero16
        return 0

    lax.fori_loop(0, slice_sz // 16, zbody, 0)
    pltpu.sync_copy(zslice_v, deg_sh.at[pl.ds(s * slice_sz, slice_sz)])
    plsc.subcore_barrier()

    def pf(cc, u):
        pltpu.async_copy(src_hbm.at[pl.ds(w * EDGES_W + cc * A1_CH, A1_CH)],
                         idx[u], semi[u])

    def wait_idx(u):
        pltpu.make_async_copy(src_hbm.at[pl.ds(0, A1_CH)], idx[u],
                              semi[u]).wait()

    def wait_sc(u):
        pltpu.make_async_copy(src_hbm.at[pl.ds(0, A1_CH)], ones_v,
                              sems[u]).wait()

    pf(0, 0)
    pf(1, 1)

    def body(c4, _):
        for u in range(4):
            cc = c4 * 4 + u

            @pl.when(cc >= 2)
            def _():
                wait_sc((u + 2) % 4)

            @pl.when(cc + 2 < A1_IT)
            def _():
                pf(cc + 2, (u + 2) % 4)

            wait_idx(u)
            pltpu.async_copy(ones_v, deg_sh.at[idx[u]], sems[u], add=True)
        return 0

    lax.fori_loop(0, A1_IT // 4, body, 0)
    wait_sc((A1_IT - 2) % 4)
    wait_sc((A1_IT - 1) % 4)
    plsc.subcore_barrier()
    pltpu.sync_copy(deg_sh.at[pl.ds(s * slice_sz, slice_sz)], zslice_v)
    pltpu.sync_copy(zslice_v,
                    deg2_hbm.at[pl.ds(c * NPAD + s * slice_sz, slice_sz)])


def _deg_hist(srcf):
    return pl.kernel(
        _a1_body,
        out_type=jax.ShapeDtypeStruct((NC * NPAD,), jnp.float32),
        mesh=_mesh(),
        compiler_params=pltpu.CompilerParams(needs_layout_passes=False),
        scratch_types=(
            [pltpu.VMEM((A1_CH,), jnp.int32)] * 4
            + [pltpu.VMEM((A1_CH,), jnp.float32)]
            + [pltpu.VMEM((NPAD // NS,), jnp.float32)]
            + [pltpu.VMEM_SHARED((NPAD,), jnp.float32)]
            + [pltpu.SemaphoreType.DMA] * 8
        ),
        name="sc_deg_hist",
    )(srcf)


K1_B = 512


def _pack2(a, b):
    ab = lax.bitcast_convert_type(a.astype(jnp.bfloat16),
                                  jnp.uint16).astype(jnp.uint32)
    bb = lax.bitcast_convert_type(b.astype(jnp.bfloat16),
                                  jnp.uint16).astype(jnp.uint32)
    return lax.bitcast_convert_type((ab << 16) | bb, jnp.int32)


def _k1_body(h_ref, fcw_ref, pv_ref, deg2_ref, z_ref, zpa_ref, zpb_ref,
             zr_ref):
    h_blk = h_ref[...]
    z = lax.dot_general(h_blk, fcw_ref[...], (((1,), (1,)), ((), ())),
                        preferred_element_type=jnp.float32)
    z_ref[...] = z
    pv = pv_ref[...]
    s4 = lax.dot_general(z, pv[0:4, :], (((1,), (1,)), ((), ())),
                         preferred_element_type=jnp.float32)
    hw = lax.dot_general(h_blk, pv[4:5, :], (((1,), (1,)), ((), ())),
                         preferred_element_type=jnp.float32)
    deg = jnp.maximum(deg2_ref[0, :] + deg2_ref[1, :], 1.0)
    feat = (hw[:, 0] * lax.rsqrt(deg))[:, None]
    zpa_ref[...] = _pack2(s4[:, 0:1], s4[:, 1:2])[:, 0]
    zpb_ref[...] = _pack2(s4[:, 2:3], feat)[:, 0]
    zr_ref[...] = s4[:, 3]


def _prep(h_pad, fc_w, pvec, deg2):
    return pl.pallas_call(
        _k1_body,
        grid=(NPAD // K1_B,),
        in_specs=[
            pl.BlockSpec((K1_B, D), lambda i: (i, 0)),
            pl.BlockSpec((D, D), lambda i: (0, 0)),
            pl.BlockSpec((8, D), lambda i: (0, 0)),
            pl.BlockSpec((NC, K1_B), lambda i: (0, i)),
        ],
        out_specs=[
            pl.BlockSpec((K1_B, D), lambda i: (i, 0)),
            pl.BlockSpec((K1_B,), lambda i: (i,)),
            pl.BlockSpec((K1_B,), lambda i: (i,)),
            pl.BlockSpec((K1_B,), lambda i: (i,)),
        ],
        out_shape=[
            jax.ShapeDtypeStruct((NPAD, D), jnp.float32),
            jax.ShapeDtypeStruct((NPAD,), jnp.int32),
            jax.ShapeDtypeStruct((NPAD,), jnp.int32),
            jax.ShapeDtypeStruct((NPAD,), jnp.float32),
        ],
        name="tc_prep",
    )(h_pad, fc_w, pvec, deg2)


def _edge_body(src_hbm, zpa_hbm, zpb_hbm, zr1_hbm, z_hbm, smalls_hbm,
               col_hbm, row_hbm, agg_hbm,
               idx0, idx1, fx0, fx1, sb0, sb1, rw0, rw1, dv0, dv1,
               cv0, cv1, rv0, rv1, ag0, ag1, sm_v,
               semi0, semi1, sems0, sems1, semr0, semr1, semd0, semd1,
               semo0, semo1):
    w = _wid()
    pltpu.sync_copy(smalls_hbm, sm_v)
    idx = [idx0, idx1]
    fx = [fx0, fx1]
    sb = [sb0, sb1]
    rw = [rw0, rw1]
    dv = [dv0, dv1]
    cv = [cv0, cv1]
    rv = [rv0, rv1]
    ag = [ag0, ag1]
    semi = [semi0, semi1]
    sems = [sems0, sems1]
    semr = [semr0, semr1]
    semd = [semd0, semd1]
    semo = [semo0, semo1]

    kio = lax.iota(jnp.int32, 16)
    klt10 = kio < K
    klt9 = kio < (K - 1)
    kcl = jnp.where(klt10, kio, K - 1)
    izero = jnp.zeros((16,), jnp.int32)
    m_hi = jnp.full((16,), -65536, jnp.int32)
    wc_vec = sm_v[...]
    wck = [wc_vec[k] for k in range(K)]
    crb = wc_vec[10]
    wbase = w * NODES_W

    def issue(c, b):
        n0 = wbase + c * CH_N
        for g0 in (0, 80):
            pltpu.async_copy(zpa_hbm.at[idx[b].at[pl.ds(g0, 80)]],
                             sb[b].at[pl.ds(g0, 80)], sems[b])
            pltpu.async_copy(zpb_hbm.at[idx[b].at[pl.ds(g0, 80)]],
                             sb[b].at[pl.ds(CH_E + g0, 80)], sems[b])
        for r0 in range(0, CH_E, 40):
            pltpu.async_copy(z_hbm.at[idx[b].at[pl.ds(r0, 40)]],
                             rw[b].at[pl.ds(r0, 40)], semr[b])
        pltpu.async_copy(zr1_hbm.at[pl.ds(n0, CH_N)], dv[b], semd[b])

    def prefetch_idx(c, b):
        pltpu.async_copy(src_hbm.at[pl.ds((wbase + c * CH_N) * K, CH_E)],
                         idx[b], semi[b])

    def wait_idx(b):
        pltpu.make_async_copy(src_hbm.at[pl.ds(0, CH_E)], idx[b],
                              semi[b]).wait()

    def wait_data(b):
        pltpu.make_async_copy(zpa_hbm.at[pl.ds(0, CH_E * 2)], sb[b],
                              sems[b]).wait()
        pltpu.make_async_copy(z_hbm.at[pl.ds(0, CH_E)], rw[b],
                              semr[b]).wait()
        pltpu.make_async_copy(zr1_hbm.at[pl.ds(0, CH_N)], dv[b],
                              semd[b]).wait()

    def wait_out(b):
        pltpu.make_async_copy(col_hbm.at[pl.ds(0, CH_N)], cv[b],
                              semo[b]).wait()
        pltpu.make_async_copy(row_hbm.at[pl.ds(0, CH_N * 16)], rv[b],
                              semo[b]).wait()
        pltpu.make_async_copy(agg_hbm.at[pl.ds(0, CH_N)], ag[b],
                              semo[b]).wait()

    def compute(c, b):
        agg_reg = jnp.zeros((16,), jnp.float32)
        for n in range(CH_N):
            base = n * K
            w0v = plsc.load_gather(sb[b], [base + kcl])
            w1v = plsc.load_gather(sb[b], [CH_E + base + kcl])
            al = plsc.bitcast(jnp.bitwise_and(w0v, m_hi), jnp.float32)
            q0 = plsc.bitcast(lax.shift_left(w0v, 16), jnp.float32)
            q1 = plsc.bitcast(jnp.bitwise_and(w1v, m_hi), jnp.float32)
            ft = plsc.bitcast(lax.shift_left(w1v, 16), jnp.float32)
            zr = plsc.load_gather(dv[b], [izero + n])
            e = al + zr
            e = jnp.where(e >= 0.0, e, 0.01 * e)
            em = jnp.where(klt10, e, -3.0e38)
            m = jnp.max(em)
            ex = jnp.where(klt10, jnp.exp(e - m), 0.0)
            alpha = ex / jnp.sum(ex)
            b1 = alpha * q1
            rv[b][pl.ds(n * 16, 16)] = b1
            b1s = plsc.load_gather(rv[b], [n * 16 + jnp.minimum(kio + 1, 15)])
            rr = alpha * q0 + b1s + crb
            rv[b][pl.ds(n * 16, 16)] = jnp.where(klt9, rr, 0.0)
            agg_reg = jnp.where(kio == n,
                                jnp.sum(jnp.where(klt10, ft, 0.0)), agg_reg)
            acc = [jnp.zeros((16,), jnp.float32) for _ in range(D // 16)]
            for k in range(K):
                bk = alpha[k] * wck[k]
                for dd in range(D // 16):
                    acc[dd] = acc[dd] + bk * rw[b][n * K + k,
                                                   pl.ds(dd * 16, 16)]
            for dd in range(D // 16):
                cv[b][n, pl.ds(dd * 16, 16)] = acc[dd]
        ag[b][...] = agg_reg
        n0 = wbase + c * CH_N
        pltpu.async_copy(cv[b], col_hbm.at[pl.ds(n0, CH_N)], semo[b])
        pltpu.async_copy(rv[b], row_hbm.at[pl.ds(n0 * 16, CH_N * 16)],
                         semo[b])
        pltpu.async_copy(ag[b], agg_hbm.at[pl.ds(n0, CH_N)], semo[b])

    pltpu.sync_copy(src_hbm.at[pl.ds(wbase * K, CH_E)], idx0)
    issue(0, 0)
    prefetch_idx(1, 1)

    def body(i2, _):
        for b in (0, 1):
            c = i2 * 2 + b
            nb = 1 - b
            nc = c + 1

            @pl.when(nc < CH_IT)
            def _():
                wait_idx(nb)
                issue(nc, nb)

            wait_data(b)

            @pl.when(nc + 1 < CH_IT)
            def _():
                prefetch_idx(nc + 1, b)

            @pl.when(c >= 2)
            def _():
                wait_out(b)

            compute(c, b)
        return 0

    lax.fori_loop(0, CH_IT // 2, body, 0)
    wait_out(0)
    wait_out(1)


def _edge_all(srcf, zpa, zpb, zr1d, z, smalls):
    return pl.kernel(
        _edge_body,
        out_type=[
            jax.ShapeDtypeStruct((NPAD, D), jnp.float32),
            jax.ShapeDtypeStruct((NPAD * 16,), jnp.float32),
            jax.ShapeDtypeStruct((NPAD,), jnp.float32),
        ],
        mesh=_mesh(),
        compiler_params=pltpu.CompilerParams(needs_layout_passes=False),
        scratch_types=(
            [pltpu.VMEM((CH_E,), jnp.int32)] * 2
            + [pltpu.VMEM((16,), jnp.int32)] * 2
            + [pltpu.VMEM((CH_E * 2,), jnp.int32)] * 2
            + [pltpu.VMEM((CH_E, D), jnp.float32)] * 2
            + [pltpu.VMEM((CH_N,), jnp.float32)] * 2
            + [pltpu.VMEM((CH_N, D), jnp.float32)] * 2
            + [pltpu.VMEM((CH_N * 16,), jnp.float32)] * 2
            + [pltpu.VMEM((CH_N,), jnp.float32)] * 2
            + [pltpu.VMEM((16,), jnp.float32)]
            + [pltpu.SemaphoreType.DMA] * 10
        ),
        name="sc_edge_all",
    )(srcf, zpa, zpb, zr1d, z, smalls)


E_B = 1024


def _e1_body(col_ref, row_ref, agg_ref, st_ref):
    i = pl.program_id(0)
    c = col_ref[...]
    s1c = jnp.sum(c)
    s2c = jnp.sum(c * c)
    rw = row_ref[...]
    r = lax.broadcasted_iota(jnp.int32, rw.shape, 0)
    l = lax.broadcasted_iota(jnp.int32, rw.shape, 1)
    node = (i * 128 + r) * 8 + jnp.right_shift(l, 4)
    rw = jnp.where(node < N, rw, 0.0)
    s1r = jnp.sum(rw)
    s2r = jnp.sum(rw * rw)
    lane = lax.broadcasted_iota(jnp.int32, (1, 128), 1)
    contrib = (jnp.where(lane == 0, s1c, 0.0)
               + jnp.where(lane == 1, s2c, 0.0)
               + jnp.where(lane == 2, s1r, 0.0)
               + jnp.where(lane == 3, s2r, 0.0))

    @pl.when(i == 0)
    def _():
        a = agg_ref[...]
        rr = lax.broadcasted_iota(jnp.int32, a.shape, 0)
        ll = lax.broadcasted_iota(jnp.int32, a.shape, 1)
        valid = rr * 128 + ll < N
        s = a * (float(K) ** -0.5)
        m = jnp.max(jnp.where(valid, s, -3.0e38))
        zsum = jnp.sum(jnp.where(valid, jnp.exp(s - m), 0.0))
        st_ref[...] = (jnp.where(lane == 4, m, 0.0)
                       + jnp.where(lane == 5, zsum, 0.0))

    st_ref[...] += contrib


def _bn_stats(col, row2d, aggd):
    return pl.pallas_call(
        _e1_body,
        grid=(NPAD // E_B,),
        in_specs=[
            pl.BlockSpec((E_B, D), lambda i: (i, 0)),
            pl.BlockSpec((E_B * 16 // 128, 128), lambda i: (i, 0)),
            pl.BlockSpec((NPAD // 128, 128), lambda i: (0, 0)),
        ],
        out_specs=pl.BlockSpec((1, 128), lambda i: (0, 0)),
        out_shape=jax.ShapeDtypeStruct((1, 128), jnp.float32),
        name="tc_bn_stats",
    )(col, row2d, aggd)


def _e2_body(col_ref, row_ref, agg_ref, h_ref, st_ref, bn_ref, l1_ref, l2_ref,
             hp_ref, cw_ref, cb_ref, out_ref, acc_ref):
    i = pl.program_id(0)
    st = st_ref[0, :]
    s1c, s2c = st[0:1], st[1:2]
    s1r, s2r = st[2:3], st[3:4]
    m, zsum = st[4:5], st[5:6]
    muc = s1c / float(N * D)
    varc = s2c / float(N * D) - muc * muc
    mur = s1r / float(N * (K - 1))
    varr = s2r / float(N * (K - 1)) - mur * mur
    gr, br = bn_ref[0, 0:1], bn_ref[1, 0:1]
    gc, bc = bn_ref[2, 0:1], bn_ref[3, 0:1]
    ac = gc * lax.rsqrt(varc + 1e-5)
    bcs = bc - muc * ac
    ar = gr * lax.rsqrt(varr + 1e-5)
    brs = br - mur * ar

    coln = jnp.maximum(col_ref[...] * ac + bcs, 0.0)
    rown = jnp.maximum(row_ref[...] * ar + brs, 0.0)
    uf = (lax.dot_general(rown, l1_ref[...], (((1,), (0,)), ((), ())),
                          preferred_element_type=jnp.float32)
          + lax.dot_general(coln, l2_ref[...], (((1,), (0,)), ((), ())),
                            preferred_element_type=jnp.float32)
          + lax.dot_general(h_ref[...], hp_ref[...], (((1,), (1,)), ((), ())),
                            preferred_element_type=jnp.float32))
    uf = jnp.maximum(uf, 0.0)
    a = agg_ref[...]
    rr = lax.broadcasted_iota(jnp.int32, a.shape, 0)
    ll = lax.broadcasted_iota(jnp.int32, a.shape, 1)
    valid = (i * 8 + rr) * 128 + ll < N
    s = a * (float(K) ** -0.5)
    wblk = jnp.where(valid, jnp.exp(s - m), 0.0) / (zsum * float(N))
    part = jnp.zeros((1, 128), jnp.float32)
    for r in range(8):
        part = part + lax.dot_general(
            wblk[r:r + 1, :], uf[r * 128:(r + 1) * 128, :],
            (((1,), (0,)), ((), ())), preferred_element_type=jnp.float32)

    @pl.when(i == 0)
    def _():
        acc_ref[...] = jnp.zeros((1, 128), jnp.float32)

    acc_ref[...] += part

    @pl.when(i == (NPAD // E_B) - 1)
    def _():
        out_ref[...] = lax.dot_general(
            acc_ref[...], cw_ref[...], (((1,), (1,)), ((), ())),
            preferred_element_type=jnp.float32) + cb_ref[...]


def _final(col, row2d, aggd, h_pad, stats, bnvec, l1p, l2, hpw, cw, cb):
    return pl.pallas_call(
        _e2_body,
        grid=(NPAD // E_B,),
        in_specs=[
            pl.BlockSpec((E_B, D), lambda i: (i, 0)),
            pl.BlockSpec((E_B, 16), lambda i: (i, 0)),
            pl.BlockSpec((8, 128), lambda i: (i, 0)),
            pl.BlockSpec((E_B, D), lambda i: (i, 0)),
            pl.BlockSpec((1, 128), lambda i: (0, 0)),
            pl.BlockSpec((8, 128), lambda i: (0, 0)),
            pl.BlockSpec((16, D), lambda i: (0, 0)),
            pl.BlockSpec((D, D), lambda i: (0, 0)),
            pl.BlockSpec((D, D), lambda i: (0, 0)),
            pl.BlockSpec((NCLS, D), lambda i: (0, 0)),
            pl.BlockSpec((1, NCLS), lambda i: (0, 0)),
        ],
        out_specs=pl.BlockSpec((1, NCLS), lambda i: (0, 0)),
        out_shape=jax.ShapeDtypeStruct((1, NCLS), jnp.float32),
        scratch_shapes=[pltpu.VMEM((1, 128), jnp.float32)],
        name="tc_final",
    )(col, row2d, aggd, h_pad, stats, bnvec, l1p, l2, hpw, cw, cb)


def kernel(h, src_idx, fc_w, attn_w, convrow_w, convrow_b, bn_row_g, bn_row_b,
           convcol_w, convcol_b, bn_col_g, bn_col_b, localw, h_proj_w,
           conv_w, conv_b, classify_w, classify_b):
    f32 = jnp.float32
    h_pad = jnp.concatenate([h, jnp.zeros((NPAD - N, D), f32)], axis=0)
    srcf = jnp.concatenate(
        [src_idx,
         jnp.full((NPAD - N, K), NPAD - 1, jnp.int32)], axis=0).reshape(-1)

    a_l = attn_w[0, :D]
    a_r = attn_w[0, D:]
    w0 = convrow_w[0, 0, 0, :]
    w1 = convrow_w[0, 0, 1, :]
    wc = convcol_w[0, 0, :, 0]
    cwv = conv_w[:, 0]
    pvec = jnp.concatenate(
        [jnp.stack([a_l, w0, w1, a_r, cwv], axis=0),
         jnp.zeros((3, D), f32)], axis=0)
    smalls = jnp.concatenate(
        [wc, convrow_b, jnp.zeros((5,), f32)], axis=0)
    bnvec = jnp.stack([
        jnp.broadcast_to(bn_row_g[0], (128,)),
        jnp.broadcast_to(bn_row_b[0], (128,)),
        jnp.broadcast_to(bn_col_g[0], (128,)),
        jnp.broadcast_to(bn_col_b[0], (128,)),
    ] + [jnp.zeros((128,), f32)] * 4, axis=0)
    l1p = jnp.concatenate(
        [localw[:K - 1, :], jnp.zeros((16 - (K - 1), D), f32)], axis=0)
    l2 = localw[K - 1:, :]
    cb = classify_b.reshape(1, NCLS)

    deg2 = _deg_hist(srcf).reshape(NC, NPAD)
    z, zpa, zpb, zr1d = _prep(h_pad, fc_w, pvec, deg2)
    col, row_f, agg = _edge_all(srcf, zpa, zpb, zr1d, z, smalls)
    aggd = agg.reshape(NPAD // 128, 128)
    rowd = row_f.reshape(NPAD * 16 // 128, 128)
    stats = _bn_stats(col, rowd, aggd)
    return _final(col, row_f.reshape(NPAD, 16), aggd, h_pad, stats, bnvec,
                  l1p, l2, h_proj_w, classify_w, cb)


# speedup vs baseline: 1.0703x; 1.0072x over previous
import functools

import jax
import jax.numpy as jnp
from jax import lax
from jax.experimental import pallas as pl
from jax.experimental.pallas import tpu as pltpu
from jax.experimental.pallas import tpu_sc as plsc

N = 50000
D = 128
K = 10
NCLS = 16

NC = 2
NS = 16
NW = NC * NS
NPAD = 50176
NODES_W = NPAD // NW
EDGES_W = NODES_W * K

A1_CH = 112
A1_IT = EDGES_W // A1_CH

CH_N = 16
CH_E = CH_N * K
CH_IT = NODES_W // CH_N

_mesh = functools.partial(plsc.VectorSubcoreMesh,
                          core_axis_name="c", subcore_axis_name="s")


def _wid():
    return lax.axis_index("c") * NS + lax.axis_index("s")


def _a1_body(src_hbm, deg2_hbm, i0, i1, i2, i3, ones_v, zslice_v, deg_sh,
             si0, si1, si2, si3, ss0, ss1, ss2, ss3):
    c = lax.axis_index("c")
    s = lax.axis_index("s")
    w = _wid()
    idx = [i0, i1, i2, i3]
    semi = [si0, si1, si2, si3]
    sems = [ss0, ss1, ss2, ss3]
    zero16 = jnp.zeros((16,), jnp.float32)
    for j in range(A1_CH // 16):
        ones_v[pl.ds(j * 16, 16)] = zero16 + 1.0

    slice_sz = NPAD // NS

    def zbody(i, _):
        zslice_v[pl.ds(i * 16, 16)] = zero16
        return 0

    lax.fori_loop(0, slice_sz // 16, zbody, 0)
    pltpu.sync_copy(zslice_v, deg_sh.at[pl.ds(s * slice_sz, slice_sz)])
    plsc.subcore_barrier()

    def pf(cc, u):
        pltpu.async_copy(src_hbm.at[pl.ds(w * EDGES_W + cc * A1_CH, A1_CH)],
                         idx[u], semi[u])

    def wait_idx(u):
        pltpu.make_async_copy(src_hbm.at[pl.ds(0, A1_CH)], idx[u],
                              semi[u]).wait()

    def wait_sc(u):
        pltpu.make_async_copy(src_hbm.at[pl.ds(0, A1_CH)], ones_v,
                              sems[u]).wait()

    pf(0, 0)
    pf(1, 1)

    def body(c4, _):
        for u in range(4):
            cc = c4 * 4 + u

            @pl.when(cc >= 2)
            def _():
                wait_sc((u + 2) % 4)

            @pl.when(cc + 2 < A1_IT)
            def _():
                pf(cc + 2, (u + 2) % 4)

            wait_idx(u)
            pltpu.async_copy(ones_v, deg_sh.at[idx[u]], sems[u], add=True)
        return 0

    lax.fori_loop(0, A1_IT // 4, body, 0)
    wait_sc((A1_IT - 2) % 4)
    wait_sc((A1_IT - 1) % 4)
    plsc.subcore_barrier()
    pltpu.sync_copy(deg_sh.at[pl.ds(s * slice_sz, slice_sz)], zslice_v)
    pltpu.sync_copy(zslice_v,
                    deg2_hbm.at[pl.ds(c * NPAD + s * slice_sz, slice_sz)])


def _deg_hist(srcf):
    return pl.kernel(
        _a1_body,
        out_type=jax.ShapeDtypeStruct((NC * NPAD,), jnp.float32),
        mesh=_mesh(),
        compiler_params=pltpu.CompilerParams(needs_layout_passes=False),
        scratch_types=(
            [pltpu.VMEM((A1_CH,), jnp.int32)] * 4
            + [pltpu.VMEM((A1_CH,), jnp.float32)]
            + [pltpu.VMEM((NPAD // NS,), jnp.float32)]
            + [pltpu.VMEM_SHARED((NPAD,), jnp.float32)]
            + [pltpu.SemaphoreType.DMA] * 8
        ),
        name="sc_deg_hist",
    )(srcf)


K1_B = 512


def _pack2(a, b):
    ab = lax.bitcast_convert_type(a.astype(jnp.bfloat16),
                                  jnp.uint16).astype(jnp.uint32)
    bb = lax.bitcast_convert_type(b.astype(jnp.bfloat16),
                                  jnp.uint16).astype(jnp.uint32)
    return lax.bitcast_convert_type((ab << 16) | bb, jnp.int32)


def _k1_body(h_ref, fcw_ref, pv_ref, deg2_ref, z_ref, zpa_ref, zpb_ref,
             zr_ref):
    h_blk = h_ref[...]
    z = lax.dot_general(h_blk, fcw_ref[...], (((1,), (1,)), ((), ())),
                        preferred_element_type=jnp.float32)
    z_ref[...] = z
    pv = pv_ref[...]
    s4 = lax.dot_general(z, pv[0:4, :], (((1,), (1,)), ((), ())),
                         preferred_element_type=jnp.float32)
    hw = lax.dot_general(h_blk, pv[4:5, :], (((1,), (1,)), ((), ())),
                         preferred_element_type=jnp.float32)
    deg = jnp.maximum(deg2_ref[0, :] + deg2_ref[1, :], 1.0)
    feat = (hw[:, 0] * lax.rsqrt(deg))[:, None]
    zpa_ref[...] = _pack2(s4[:, 0:1], s4[:, 1:2])[:, 0]
    zpb_ref[...] = _pack2(s4[:, 2:3], feat)[:, 0]
    zr_ref[...] = s4[:, 3]


def _prep(h_pad, fc_w, pvec, deg2):
    return pl.pallas_call(
        _k1_body,
        grid=(NPAD // K1_B,),
        in_specs=[
            pl.BlockSpec((K1_B, D), lambda i: (i, 0)),
            pl.BlockSpec((D, D), lambda i: (0, 0)),
            pl.BlockSpec((8, D), lambda i: (0, 0)),
            pl.BlockSpec((NC, K1_B), lambda i: (0, i)),
        ],
        out_specs=[
            pl.BlockSpec((K1_B, D), lambda i: (i, 0)),
            pl.BlockSpec((K1_B,), lambda i: (i,)),
            pl.BlockSpec((K1_B,), lambda i: (i,)),
            pl.BlockSpec((K1_B,), lambda i: (i,)),
        ],
        out_shape=[
            jax.ShapeDtypeStruct((NPAD, D), jnp.float32),
            jax.ShapeDtypeStruct((NPAD,), jnp.int32),
            jax.ShapeDtypeStruct((NPAD,), jnp.int32),
            jax.ShapeDtypeStruct((NPAD,), jnp.float32),
        ],
        name="tc_prep",
    )(h_pad, fc_w, pvec, deg2)


def _edge_body(src_hbm, zpa_hbm, zpb_hbm, zr1_hbm, z_hbm, smalls_hbm,
               col_hbm, row_hbm, agg_hbm, part_hbm,
               idx0, idx1, fx0, fx1, sb0, sb1, rw0, rw1, dv0, dv1,
               cv0, cv1, rv0, rv1, ag0, ag1, sm_v,
               semi0, semi1, sems0, sems1, semr0, semr1, semd0, semd1,
               semo0, semo1):
    w = _wid()
    pltpu.sync_copy(smalls_hbm, sm_v)
    idx = [idx0, idx1]
    fx = [fx0, fx1]
    sb = [sb0, sb1]
    rw = [rw0, rw1]
    dv = [dv0, dv1]
    cv = [cv0, cv1]
    rv = [rv0, rv1]
    ag = [ag0, ag1]
    semi = [semi0, semi1]
    sems = [sems0, sems1]
    semr = [semr0, semr1]
    semd = [semd0, semd1]
    semo = [semo0, semo1]

    kio = lax.iota(jnp.int32, 16)
    klt10 = kio < K
    klt9 = kio < (K - 1)
    kcl = jnp.where(klt10, kio, K - 1)
    izero = jnp.zeros((16,), jnp.int32)
    m_hi = jnp.full((16,), -65536, jnp.int32)
    wc_vec = sm_v[...]
    wck = [wc_vec[k] for k in range(K)]
    crb = wc_vec[10]
    wbase = w * NODES_W

    def issue(c, b):
        n0 = wbase + c * CH_N
        for g0 in (0, 80):
            pltpu.async_copy(zpa_hbm.at[idx[b].at[pl.ds(g0, 80)]],
                             sb[b].at[pl.ds(g0, 80)], sems[b])
            pltpu.async_copy(zpb_hbm.at[idx[b].at[pl.ds(g0, 80)]],
                             sb[b].at[pl.ds(CH_E + g0, 80)], sems[b])
        for r0 in range(0, CH_E, 40):
            pltpu.async_copy(z_hbm.at[idx[b].at[pl.ds(r0, 40)]],
                             rw[b].at[pl.ds(r0, 40)], semr[b])
        pltpu.async_copy(zr1_hbm.at[pl.ds(n0, CH_N)], dv[b], semd[b])

    def prefetch_idx(c, b):
        pltpu.async_copy(src_hbm.at[pl.ds((wbase + c * CH_N) * K, CH_E)],
                         idx[b], semi[b])

    def wait_idx(b):
        pltpu.make_async_copy(src_hbm.at[pl.ds(0, CH_E)], idx[b],
                              semi[b]).wait()

    def wait_data(b):
        pltpu.make_async_copy(zpa_hbm.at[pl.ds(0, CH_E * 2)], sb[b],
                              sems[b]).wait()
        pltpu.make_async_copy(z_hbm.at[pl.ds(0, CH_E)], rw[b],
                              semr[b]).wait()
        pltpu.make_async_copy(zr1_hbm.at[pl.ds(0, CH_N)], dv[b],
                              semd[b]).wait()

    def wait_out(b):
        pltpu.make_async_copy(col_hbm.at[pl.ds(0, CH_N)], cv[b],
                              semo[b]).wait()
        pltpu.make_async_copy(row_hbm.at[pl.ds(0, CH_N * 16)], rv[b],
                              semo[b]).wait()
        pltpu.make_async_copy(agg_hbm.at[pl.ds(0, CH_N)], ag[b],
                              semo[b]).wait()

    def compute(c, b, stat_regs):
        zv = jnp.zeros((16,), jnp.float32)
        vs1c, vs2c, vs1r, vs2r = zv, zv, zv, zv
        n0c = wbase + c * CH_N
        agg_reg = jnp.zeros((16,), jnp.float32)
        for n in range(CH_N):
            base = n * K
            w0v = plsc.load_gather(sb[b], [base + kcl])
            w1v = plsc.load_gather(sb[b], [CH_E + base + kcl])
            al = plsc.bitcast(jnp.bitwise_and(w0v, m_hi), jnp.float32)
            q0 = plsc.bitcast(lax.shift_left(w0v, 16), jnp.float32)
            q1 = plsc.bitcast(jnp.bitwise_and(w1v, m_hi), jnp.float32)
            ft = plsc.bitcast(lax.shift_left(w1v, 16), jnp.float32)
            zr = plsc.load_gather(dv[b], [izero + n])
            e = al + zr
            e = jnp.where(e >= 0.0, e, 0.01 * e)
            em = jnp.where(klt10, e, -3.0e38)
            m = jnp.max(em)
            ex = jnp.where(klt10, jnp.exp(e - m), 0.0)
            alpha = ex / jnp.sum(ex)
            b1 = alpha * q1
            rv[b][pl.ds(n * 16, 16)] = b1
            b1s = plsc.load_gather(rv[b], [n * 16 + jnp.minimum(kio + 1, 15)])
            rr = alpha * q0 + b1s + crb
            rrm = jnp.where(klt9, rr, 0.0)
            rv[b][pl.ds(n * 16, 16)] = rrm
            rvalid = jnp.where(n0c + n < N, 1.0, 0.0)
            vs1r = vs1r + rrm * rvalid
            vs2r = vs2r + (rrm * rrm) * rvalid
            agg_reg = jnp.where(kio == n,
                                jnp.sum(jnp.where(klt10, ft, 0.0)), agg_reg)
            acc = [jnp.zeros((16,), jnp.float32) for _ in range(D // 16)]
            for k in range(K):
                bk = alpha[k] * wck[k]
                for dd in range(D // 16):
                    acc[dd] = acc[dd] + bk * rw[b][n * K + k,
                                                   pl.ds(dd * 16, 16)]
            for dd in range(D // 16):
                cv[b][n, pl.ds(dd * 16, 16)] = acc[dd]
                vs1c = vs1c + acc[dd]
                vs2c = vs2c + acc[dd] * acc[dd]
        ag[b][...] = agg_reg
        n0 = wbase + c * CH_N
        pltpu.async_copy(cv[b], col_hbm.at[pl.ds(n0, CH_N)], semo[b])
        pltpu.async_copy(rv[b], row_hbm.at[pl.ds(n0 * 16, CH_N * 16)],
                         semo[b])
        pltpu.async_copy(ag[b], agg_hbm.at[pl.ds(n0, CH_N)], semo[b])
        t1c, t2c, t1r, t2r = stat_regs
        return (t1c + vs1c, t2c + vs2c, t1r + vs1r, t2r + vs2r)

    pltpu.sync_copy(src_hbm.at[pl.ds(wbase * K, CH_E)], idx0)
    issue(0, 0)
    prefetch_idx(1, 1)

    def body(i2, stat_regs):
        for b in (0, 1):
            c = i2 * 2 + b
            nb = 1 - b
            nc = c + 1

            @pl.when(nc < CH_IT)
            def _():
                wait_idx(nb)
                issue(nc, nb)

            wait_data(b)

            @pl.when(nc + 1 < CH_IT)
            def _():
                prefetch_idx(nc + 1, b)

            @pl.when(c >= 2)
            def _():
                wait_out(b)

            stat_regs = compute(c, b, stat_regs)
        return stat_regs

    z16 = jnp.zeros((16,), jnp.float32)
    vs1c, vs2c, vs1r, vs2r = lax.fori_loop(0, CH_IT // 2, body,
                                           (z16, z16, z16, z16))
    wait_out(0)
    wait_out(1)
    sums = (jnp.where(kio == 0, jnp.sum(vs1c), 0.0)
            + jnp.where(kio == 1, jnp.sum(vs2c), 0.0)
            + jnp.where(kio == 2, jnp.sum(vs1r), 0.0)
            + jnp.where(kio == 3, jnp.sum(vs2r), 0.0))
    ag[0][...] = sums
    pltpu.sync_copy(ag[0], part_hbm.at[pl.ds(w * 16, 16)])


def _edge_all(srcf, zpa, zpb, zr1d, z, smalls):
    return pl.kernel(
        _edge_body,
        out_type=[
            jax.ShapeDtypeStruct((NPAD, D), jnp.float32),
            jax.ShapeDtypeStruct((NPAD * 16,), jnp.float32),
            jax.ShapeDtypeStruct((NPAD,), jnp.float32),
            jax.ShapeDtypeStruct((NW * 16,), jnp.float32),
        ],
        mesh=_mesh(),
        compiler_params=pltpu.CompilerParams(needs_layout_passes=False),
        scratch_types=(
            [pltpu.VMEM((CH_E,), jnp.int32)] * 2
            + [pltpu.VMEM((16,), jnp.int32)] * 2
            + [pltpu.VMEM((CH_E * 2,), jnp.int32)] * 2
            + [pltpu.VMEM((CH_E, D), jnp.float32)] * 2
            + [pltpu.VMEM((CH_N,), jnp.float32)] * 2
            + [pltpu.VMEM((CH_N, D), jnp.float32)] * 2
            + [pltpu.VMEM((CH_N * 16,), jnp.float32)] * 2
            + [pltpu.VMEM((CH_N,), jnp.float32)] * 2
            + [pltpu.VMEM((16,), jnp.float32)]
            + [pltpu.SemaphoreType.DMA] * 10
        ),
        name="sc_edge_all",
    )(srcf, zpa, zpb, zr1d, z, smalls)


E_B = 1024


def _e1_body(part_ref, agg_ref, st_ref):
    p = part_ref[...]
    lane = lax.broadcasted_iota(jnp.int32, (1, 128), 1)
    pl16 = jnp.bitwise_and(lax.broadcasted_iota(jnp.int32, p.shape, 1), 15)
    s1c = jnp.sum(jnp.where(pl16 == 0, p, 0.0))
    s2c = jnp.sum(jnp.where(pl16 == 1, p, 0.0))
    s1r = jnp.sum(jnp.where(pl16 == 2, p, 0.0))
    s2r = jnp.sum(jnp.where(pl16 == 3, p, 0.0))
    a = agg_ref[...]
    rr = lax.broadcasted_iota(jnp.int32, a.shape, 0)
    ll = lax.broadcasted_iota(jnp.int32, a.shape, 1)
    valid = rr * 128 + ll < N
    s = a * (float(K) ** -0.5)
    m = jnp.max(jnp.where(valid, s, -3.0e38))
    zsum = jnp.sum(jnp.where(valid, jnp.exp(s - m), 0.0))
    st_ref[...] = (jnp.where(lane == 0, s1c, 0.0)
                   + jnp.where(lane == 1, s2c, 0.0)
                   + jnp.where(lane == 2, s1r, 0.0)
                   + jnp.where(lane == 3, s2r, 0.0)
                   + jnp.where(lane == 4, m, 0.0)
                   + jnp.where(lane == 5, zsum, 0.0))


def _bn_stats(partd, aggd):
    return pl.pallas_call(
        _e1_body,
        grid=(1,),
        in_specs=[
            pl.BlockSpec((NW * 16 // 128, 128), lambda i: (0, 0)),
            pl.BlockSpec((NPAD // 128, 128), lambda i: (0, 0)),
        ],
        out_specs=pl.BlockSpec((1, 128), lambda i: (0, 0)),
        out_shape=jax.ShapeDtypeStruct((1, 128), jnp.float32),
        name="tc_bn_stats",
    )(partd, aggd)


def _e2_body(col_ref, row_ref, agg_ref, h_ref, st_ref, bn_ref, l1_ref, l2_ref,
             hp_ref, cw_ref, cb_ref, out_ref, acc_ref):
    i = pl.program_id(0)
    st = st_ref[0, :]
    s1c, s2c = st[0:1], st[1:2]
    s1r, s2r = st[2:3], st[3:4]
    m, zsum = st[4:5], st[5:6]
    muc = s1c / float(N * D)
    varc = s2c / float(N * D) - muc * muc
    mur = s1r / float(N * (K - 1))
    varr = s2r / float(N * (K - 1)) - mur * mur
    gr, br = bn_ref[0, 0:1], bn_ref[1, 0:1]
    gc, bc = bn_ref[2, 0:1], bn_ref[3, 0:1]
    ac = gc * lax.rsqrt(varc + 1e-5)
    bcs = bc - muc * ac
    ar = gr * lax.rsqrt(varr + 1e-5)
    brs = br - mur * ar

    coln = jnp.maximum(col_ref[...] * ac + bcs, 0.0)
    rown = jnp.maximum(row_ref[...] * ar + brs, 0.0)
    uf = (lax.dot_general(rown, l1_ref[...], (((1,), (0,)), ((), ())),
                          preferred_element_type=jnp.float32)
          + lax.dot_general(coln, l2_ref[...], (((1,), (0,)), ((), ())),
                            preferred_element_type=jnp.float32)
          + lax.dot_general(h_ref[...], hp_ref[...], (((1,), (1,)), ((), ())),
                            preferred_element_type=jnp.float32))
    uf = jnp.maximum(uf, 0.0)
    a = agg_ref[...]
    rr = lax.broadcasted_iota(jnp.int32, a.shape, 0)
    ll = lax.broadcasted_iota(jnp.int32, a.shape, 1)
    valid = (i * 8 + rr) * 128 + ll < N
    s = a * (float(K) ** -0.5)
    wblk = jnp.where(valid, jnp.exp(s - m), 0.0) / (zsum * float(N))
    part = jnp.zeros((1, 128), jnp.float32)
    for r in range(8):
        part = part + lax.dot_general(
            wblk[r:r + 1, :], uf[r * 128:(r + 1) * 128, :],
            (((1,), (0,)), ((), ())), preferred_element_type=jnp.float32)

    @pl.when(i == 0)
    def _():
        acc_ref[...] = jnp.zeros((1, 128), jnp.float32)

    acc_ref[...] += part

    @pl.when(i == (NPAD // E_B) - 1)
    def _():
        out_ref[...] = lax.dot_general(
            acc_ref[...], cw_ref[...], (((1,), (1,)), ((), ())),
            preferred_element_type=jnp.float32) + cb_ref[...]


def _final(col, row2d, aggd, h_pad, stats, bnvec, l1p, l2, hpw, cw, cb):
    return pl.pallas_call(
        _e2_body,
        grid=(NPAD // E_B,),
        in_specs=[
            pl.BlockSpec((E_B, D), lambda i: (i, 0)),
            pl.BlockSpec((E_B, 16), lambda i: (i, 0)),
            pl.BlockSpec((8, 128), lambda i: (i, 0)),
            pl.BlockSpec((E_B, D), lambda i: (i, 0)),
            pl.BlockSpec((1, 128), lambda i: (0, 0)),
            pl.BlockSpec((8, 128), lambda i: (0, 0)),
            pl.BlockSpec((16, D), lambda i: (0, 0)),
            pl.BlockSpec((D, D), lambda i: (0, 0)),
            pl.BlockSpec((D, D), lambda i: (0, 0)),
            pl.BlockSpec((NCLS, D), lambda i: (0, 0)),
            pl.BlockSpec((1, NCLS), lambda i: (0, 0)),
        ],
        out_specs=pl.BlockSpec((1, NCLS), lambda i: (0, 0)),
        out_shape=jax.ShapeDtypeStruct((1, NCLS), jnp.float32),
        scratch_shapes=[pltpu.VMEM((1, 128), jnp.float32)],
        name="tc_final",
    )(col, row2d, aggd, h_pad, stats, bnvec, l1p, l2, hpw, cw, cb)


def kernel(h, src_idx, fc_w, attn_w, convrow_w, convrow_b, bn_row_g, bn_row_b,
           convcol_w, convcol_b, bn_col_g, bn_col_b, localw, h_proj_w,
           conv_w, conv_b, classify_w, classify_b):
    f32 = jnp.float32
    h_pad = jnp.concatenate([h, jnp.zeros((NPAD - N, D), f32)], axis=0)
    srcf = jnp.concatenate(
        [src_idx,
         jnp.full((NPAD - N, K), NPAD - 1, jnp.int32)], axis=0).reshape(-1)

    a_l = attn_w[0, :D]
    a_r = attn_w[0, D:]
    w0 = convrow_w[0, 0, 0, :]
    w1 = convrow_w[0, 0, 1, :]
    wc = convcol_w[0, 0, :, 0]
    cwv = conv_w[:, 0]
    pvec = jnp.concatenate(
        [jnp.stack([a_l, w0, w1, a_r, cwv], axis=0),
         jnp.zeros((3, D), f32)], axis=0)
    smalls = jnp.concatenate(
        [wc, convrow_b, jnp.zeros((5,), f32)], axis=0)
    bnvec = jnp.stack([
        jnp.broadcast_to(bn_row_g[0], (128,)),
        jnp.broadcast_to(bn_row_b[0], (128,)),
        jnp.broadcast_to(bn_col_g[0], (128,)),
        jnp.broadcast_to(bn_col_b[0], (128,)),
    ] + [jnp.zeros((128,), f32)] * 4, axis=0)
    l1p = jnp.concatenate(
        [localw[:K - 1, :], jnp.zeros((16 - (K - 1), D), f32)], axis=0)
    l2 = localw[K - 1:, :]
    cb = classify_b.reshape(1, NCLS)

    deg2 = _deg_hist(srcf).reshape(NC, NPAD)
    z, zpa, zpb, zr1d = _prep(h_pad, fc_w, pvec, deg2)
    col, row_f, agg, part = _edge_all(srcf, zpa, zpb, zr1d, z, smalls)
    aggd = agg.reshape(NPAD // 128, 128)
    stats = _bn_stats(part.reshape(NW * 16 // 128, 128), aggd)
    return _final(col, row_f.reshape(NPAD, 16), aggd, h_pad, stats, bnvec,
                  l1p, l2, h_proj_w, classify_w, cb)


# speedup vs baseline: 1.0841x; 1.0129x over previous
import functools

import jax
import jax.numpy as jnp
from jax import lax
from jax.experimental import pallas as pl
from jax.experimental.pallas import tpu as pltpu
from jax.experimental.pallas import tpu_sc as plsc

N = 50000
D = 128
K = 10
NCLS = 16

NC = 2
NS = 16
NW = NC * NS
NPAD = 50176
NODES_W = NPAD // NW
EDGES_W = NODES_W * K

A1_CH = 112
A1_IT = EDGES_W // A1_CH

CH_N = 16
CH_E = CH_N * K
CH_IT = NODES_W // CH_N

_mesh = functools.partial(plsc.VectorSubcoreMesh,
                          core_axis_name="c", subcore_axis_name="s")


def _wid():
    return lax.axis_index("c") * NS + lax.axis_index("s")


def _a1_body(src_hbm, deg2_hbm, i0, i1, i2, i3, ones_v, zslice_v, deg_sh,
             si0, si1, si2, si3, ss0, ss1, ss2, ss3):
    c = lax.axis_index("c")
    s = lax.axis_index("s")
    w = _wid()
    idx = [i0, i1, i2, i3]
    semi = [si0, si1, si2, si3]
    sems = [ss0, ss1, ss2, ss3]
    zero16 = jnp.zeros((16,), jnp.float32)
    for j in range(A1_CH // 16):
        ones_v[pl.ds(j * 16, 16)] = zero16 + 1.0

    slice_sz = NPAD // NS

    def zbody(i, _):
        zslice_v[pl.ds(i * 16, 16)] = zero16
        return 0

    lax.fori_loop(0, slice_sz // 16, zbody, 0)
    pltpu.sync_copy(zslice_v, deg_sh.at[pl.ds(s * slice_sz, slice_sz)])
    plsc.subcore_barrier()

    def pf(cc, u):
        pltpu.async_copy(src_hbm.at[pl.ds(w * EDGES_W + cc * A1_CH, A1_CH)],
                         idx[u], semi[u])

    def wait_idx(u):
        pltpu.make_async_copy(src_hbm.at[pl.ds(0, A1_CH)], idx[u],
                              semi[u]).wait()

    def wait_sc(u):
        pltpu.make_async_copy(src_hbm.at[pl.ds(0, A1_CH)], ones_v,
                              sems[u]).wait()

    pf(0, 0)
    pf(1, 1)

    def body(c4, _):
        for u in range(4):
            cc = c4 * 4 + u

            @pl.when(cc >= 2)
            def _():
                wait_sc((u + 2) % 4)

            @pl.when(cc + 2 < A1_IT)
            def _():
                pf(cc + 2, (u + 2) % 4)

            wait_idx(u)
            pltpu.async_copy(ones_v, deg_sh.at[idx[u]], sems[u], add=True)
        return 0

    lax.fori_loop(0, A1_IT // 4, body, 0)
    wait_sc((A1_IT - 2) % 4)
    wait_sc((A1_IT - 1) % 4)
    plsc.subcore_barrier()
    pltpu.sync_copy(deg_sh.at[pl.ds(s * slice_sz, slice_sz)], zslice_v)
    pltpu.sync_copy(zslice_v,
                    deg2_hbm.at[pl.ds(c * NPAD + s * slice_sz, slice_sz)])


def _deg_hist(srcf):
    return pl.kernel(
        _a1_body,
        out_type=jax.ShapeDtypeStruct((NC * NPAD,), jnp.float32),
        mesh=_mesh(),
        compiler_params=pltpu.CompilerParams(needs_layout_passes=False),
        scratch_types=(
            [pltpu.VMEM((A1_CH,), jnp.int32)] * 4
            + [pltpu.VMEM((A1_CH,), jnp.float32)]
            + [pltpu.VMEM((NPAD // NS,), jnp.float32)]
            + [pltpu.VMEM_SHARED((NPAD,), jnp.float32)]
            + [pltpu.SemaphoreType.DMA] * 8
        ),
        name="sc_deg_hist",
    )(srcf)


K1_B = 1024


def _pack2(a, b):
    ab = lax.bitcast_convert_type(a.astype(jnp.bfloat16),
                                  jnp.uint16).astype(jnp.uint32)
    bb = lax.bitcast_convert_type(b.astype(jnp.bfloat16),
                                  jnp.uint16).astype(jnp.uint32)
    return lax.bitcast_convert_type((ab << 16) | bb, jnp.int32)


def _k1_body(h_ref, fcw_ref, pv_ref, deg2_ref, z_ref, zpa_ref, zpb_ref,
             zr_ref):
    h_blk = h_ref[...]
    z = lax.dot_general(h_blk, fcw_ref[...], (((1,), (1,)), ((), ())),
                        preferred_element_type=jnp.float32)
    z_ref[...] = z
    pv = pv_ref[...]
    s4 = lax.dot_general(z, pv[0:4, :], (((1,), (1,)), ((), ())),
                         preferred_element_type=jnp.float32)
    hw = lax.dot_general(h_blk, pv[4:5, :], (((1,), (1,)), ((), ())),
                         preferred_element_type=jnp.float32)
    deg = jnp.maximum(deg2_ref[0, :] + deg2_ref[1, :], 1.0)
    feat = (hw[:, 0] * lax.rsqrt(deg))[:, None]
    zpa_ref[...] = _pack2(s4[:, 0:1], s4[:, 1:2])[:, 0]
    zpb_ref[...] = _pack2(s4[:, 2:3], feat)[:, 0]
    zr_ref[...] = s4[:, 3]


def _prep(h_pad, fc_w, pvec, deg2):
    return pl.pallas_call(
        _k1_body,
        grid=(NPAD // K1_B,),
        in_specs=[
            pl.BlockSpec((K1_B, D), lambda i: (i, 0)),
            pl.BlockSpec((D, D), lambda i: (0, 0)),
            pl.BlockSpec((8, D), lambda i: (0, 0)),
            pl.BlockSpec((NC, K1_B), lambda i: (0, i)),
        ],
        out_specs=[
            pl.BlockSpec((K1_B, D), lambda i: (i, 0)),
            pl.BlockSpec((K1_B,), lambda i: (i,)),
            pl.BlockSpec((K1_B,), lambda i: (i,)),
            pl.BlockSpec((K1_B,), lambda i: (i,)),
        ],
        out_shape=[
            jax.ShapeDtypeStruct((NPAD, D), jnp.float32),
            jax.ShapeDtypeStruct((NPAD,), jnp.int32),
            jax.ShapeDtypeStruct((NPAD,), jnp.int32),
            jax.ShapeDtypeStruct((NPAD,), jnp.float32),
        ],
        name="tc_prep",
    )(h_pad, fc_w, pvec, deg2)


def _edge_body(src_hbm, zpa_hbm, zpb_hbm, zr1_hbm, z_hbm, smalls_hbm,
               col_hbm, row_hbm, agg_hbm, part_hbm,
               idx0, idx1, fx0, fx1, sb0, sb1, rw0, rw1, dv0, dv1,
               cv0, cv1, rv0, rv1, ag0, ag1, sm_v,
               semi0, semi1, sems0, sems1, semr0, semr1, semd0, semd1,
               semo0, semo1):
    w = _wid()
    pltpu.sync_copy(smalls_hbm, sm_v)
    idx = [idx0, idx1]
    fx = [fx0, fx1]
    sb = [sb0, sb1]
    rw = [rw0, rw1]
    dv = [dv0, dv1]
    cv = [cv0, cv1]
    rv = [rv0, rv1]
    ag = [ag0, ag1]
    semi = [semi0, semi1]
    sems = [sems0, sems1]
    semr = [semr0, semr1]
    semd = [semd0, semd1]
    semo = [semo0, semo1]

    kio = lax.iota(jnp.int32, 16)
    klt10 = kio < K
    klt9 = kio < (K - 1)
    kcl = jnp.where(klt10, kio, K - 1)
    izero = jnp.zeros((16,), jnp.int32)
    m_hi = jnp.full((16,), -65536, jnp.int32)
    wc_vec = sm_v[...]
    wck = [wc_vec[k] for k in range(K)]
    crb = wc_vec[10]
    wbase = w * NODES_W

    def issue(c, b):
        n0 = wbase + c * CH_N
        for g0 in (0, 80):
            pltpu.async_copy(zpa_hbm.at[idx[b].at[pl.ds(g0, 80)]],
                             sb[b].at[pl.ds(g0, 80)], sems[b])
            pltpu.async_copy(zpb_hbm.at[idx[b].at[pl.ds(g0, 80)]],
                             sb[b].at[pl.ds(CH_E + g0, 80)], sems[b])
        for r0 in range(0, CH_E, 40):
            pltpu.async_copy(z_hbm.at[idx[b].at[pl.ds(r0, 40)]],
                             rw[b].at[pl.ds(r0, 40)], semr[b])
        pltpu.async_copy(zr1_hbm.at[pl.ds(n0, CH_N)], dv[b], semd[b])

    def prefetch_idx(c, b):
        pltpu.async_copy(src_hbm.at[pl.ds((wbase + c * CH_N) * K, CH_E)],
                         idx[b], semi[b])

    def wait_idx(b):
        pltpu.make_async_copy(src_hbm.at[pl.ds(0, CH_E)], idx[b],
                              semi[b]).wait()

    def wait_data(b):
        pltpu.make_async_copy(zpa_hbm.at[pl.ds(0, CH_E * 2)], sb[b],
                              sems[b]).wait()
        pltpu.make_async_copy(z_hbm.at[pl.ds(0, CH_E)], rw[b],
                              semr[b]).wait()
        pltpu.make_async_copy(zr1_hbm.at[pl.ds(0, CH_N)], dv[b],
                              semd[b]).wait()

    def wait_out(b):
        pltpu.make_async_copy(col_hbm.at[pl.ds(0, CH_N)], cv[b],
                              semo[b]).wait()
        pltpu.make_async_copy(row_hbm.at[pl.ds(0, CH_N * 16)], rv[b],
                              semo[b]).wait()
        pltpu.make_async_copy(agg_hbm.at[pl.ds(0, CH_N)], ag[b],
                              semo[b]).wait()

    def compute(c, b, stat_regs):
        zv = jnp.zeros((16,), jnp.float32)
        vs1c, vs2c, vs1r, vs2r = zv, zv, zv, zv
        n0c = wbase + c * CH_N
        agg_reg = jnp.zeros((16,), jnp.float32)
        for n in range(CH_N):
            base = n * K
            w0v = plsc.load_gather(sb[b], [base + kcl])
            w1v = plsc.load_gather(sb[b], [CH_E + base + kcl])
            al = plsc.bitcast(jnp.bitwise_and(w0v, m_hi), jnp.float32)
            q0 = plsc.bitcast(lax.shift_left(w0v, 16), jnp.float32)
            q1 = plsc.bitcast(jnp.bitwise_and(w1v, m_hi), jnp.float32)
            ft = plsc.bitcast(lax.shift_left(w1v, 16), jnp.float32)
            zr = plsc.load_gather(dv[b], [izero + n])
            e = al + zr
            e = jnp.where(e >= 0.0, e, 0.01 * e)
            em = jnp.where(klt10, e, -3.0e38)
            m = jnp.max(em)
            ex = jnp.where(klt10, jnp.exp(e - m), 0.0)
            alpha = ex / jnp.sum(ex)
            b1 = alpha * q1
            rv[b][pl.ds(n * 16, 16)] = b1
            b1s = plsc.load_gather(rv[b], [n * 16 + jnp.minimum(kio + 1, 15)])
            rr = alpha * q0 + b1s + crb
            rrm = jnp.where(klt9, rr, 0.0)
            rv[b][pl.ds(n * 16, 16)] = rrm
            rvalid = jnp.where(n0c + n < N, 1.0, 0.0)
            vs1r = vs1r + rrm * rvalid
            vs2r = vs2r + (rrm * rrm) * rvalid
            agg_reg = jnp.where(kio == n,
                                jnp.sum(jnp.where(klt10, ft, 0.0)), agg_reg)
            acc = [jnp.zeros((16,), jnp.float32) for _ in range(D // 16)]
            for k in range(K):
                bk = alpha[k] * wck[k]
                for dd in range(D // 16):
                    acc[dd] = acc[dd] + bk * rw[b][n * K + k,
                                                   pl.ds(dd * 16, 16)]
            for dd in range(D // 16):
                cv[b][n, pl.ds(dd * 16, 16)] = acc[dd]
                vs1c = vs1c + acc[dd]
                vs2c = vs2c + acc[dd] * acc[dd]
        ag[b][...] = agg_reg
        n0 = wbase + c * CH_N
        pltpu.async_copy(cv[b], col_hbm.at[pl.ds(n0, CH_N)], semo[b])
        pltpu.async_copy(rv[b], row_hbm.at[pl.ds(n0 * 16, CH_N * 16)],
                         semo[b])
        pltpu.async_copy(ag[b], agg_hbm.at[pl.ds(n0, CH_N)], semo[b])
        t1c, t2c, t1r, t2r = stat_regs
        return (t1c + vs1c, t2c + vs2c, t1r + vs1r, t2r + vs2r)

    pltpu.sync_copy(src_hbm.at[pl.ds(wbase * K, CH_E)], idx0)
    issue(0, 0)
    prefetch_idx(1, 1)

    def body(i2, stat_regs):
        for b in (0, 1):
            c = i2 * 2 + b
            nb = 1 - b
            nc = c + 1

            @pl.when(nc < CH_IT)
            def _():
                wait_idx(nb)
                issue(nc, nb)

            wait_data(b)

            @pl.when(nc + 1 < CH_IT)
            def _():
                prefetch_idx(nc + 1, b)

            @pl.when(c >= 2)
            def _():
                wait_out(b)

            stat_regs = compute(c, b, stat_regs)
        return stat_regs

    z16 = jnp.zeros((16,), jnp.float32)
    vs1c, vs2c, vs1r, vs2r = lax.fori_loop(0, CH_IT // 2, body,
                                           (z16, z16, z16, z16))
    wait_out(0)
    wait_out(1)
    sums = (jnp.where(kio == 0, jnp.sum(vs1c), 0.0)
            + jnp.where(kio == 1, jnp.sum(vs2c), 0.0)
            + jnp.where(kio == 2, jnp.sum(vs1r), 0.0)
            + jnp.where(kio == 3, jnp.sum(vs2r), 0.0))
    ag[0][...] = sums
    pltpu.sync_copy(ag[0], part_hbm.at[pl.ds(w * 16, 16)])


def _edge_all(srcf, zpa, zpb, zr1d, z, smalls):
    return pl.kernel(
        _edge_body,
        out_type=[
            jax.ShapeDtypeStruct((NPAD, D), jnp.float32),
            jax.ShapeDtypeStruct((NPAD * 16,), jnp.float32),
            jax.ShapeDtypeStruct((NPAD,), jnp.float32),
            jax.ShapeDtypeStruct((NW * 16,), jnp.float32),
        ],
        mesh=_mesh(),
        compiler_params=pltpu.CompilerParams(needs_layout_passes=False),
        scratch_types=(
            [pltpu.VMEM((CH_E,), jnp.int32)] * 2
            + [pltpu.VMEM((16,), jnp.int32)] * 2
            + [pltpu.VMEM((CH_E * 2,), jnp.int32)] * 2
            + [pltpu.VMEM((CH_E, D), jnp.float32)] * 2
            + [pltpu.VMEM((CH_N,), jnp.float32)] * 2
            + [pltpu.VMEM((CH_N, D), jnp.float32)] * 2
            + [pltpu.VMEM((CH_N * 16,), jnp.float32)] * 2
            + [pltpu.VMEM((CH_N,), jnp.float32)] * 2
            + [pltpu.VMEM((16,), jnp.float32)]
            + [pltpu.SemaphoreType.DMA] * 10
        ),
        name="sc_edge_all",
    )(srcf, zpa, zpb, zr1d, z, smalls)


E_B = 1024


def _e1_body(part_ref, agg_ref, st_ref):
    p = part_ref[...]
    lane = lax.broadcasted_iota(jnp.int32, (1, 128), 1)
    pl16 = jnp.bitwise_and(lax.broadcasted_iota(jnp.int32, p.shape, 1), 15)
    s1c = jnp.sum(jnp.where(pl16 == 0, p, 0.0))
    s2c = jnp.sum(jnp.where(pl16 == 1, p, 0.0))
    s1r = jnp.sum(jnp.where(pl16 == 2, p, 0.0))
    s2r = jnp.sum(jnp.where(pl16 == 3, p, 0.0))
    a = agg_ref[...]
    rr = lax.broadcasted_iota(jnp.int32, a.shape, 0)
    ll = lax.broadcasted_iota(jnp.int32, a.shape, 1)
    valid = rr * 128 + ll < N
    s = a * (float(K) ** -0.5)
    m = jnp.max(jnp.where(valid, s, -3.0e38))
    zsum = jnp.sum(jnp.where(valid, jnp.exp(s - m), 0.0))
    st_ref[...] = (jnp.where(lane == 0, s1c, 0.0)
                   + jnp.where(lane == 1, s2c, 0.0)
                   + jnp.where(lane == 2, s1r, 0.0)
                   + jnp.where(lane == 3, s2r, 0.0)
                   + jnp.where(lane == 4, m, 0.0)
                   + jnp.where(lane == 5, zsum, 0.0))


def _bn_stats(partd, aggd):
    return pl.pallas_call(
        _e1_body,
        grid=(1,),
        in_specs=[
            pl.BlockSpec((NW * 16 // 128, 128), lambda i: (0, 0)),
            pl.BlockSpec((NPAD // 128, 128), lambda i: (0, 0)),
        ],
        out_specs=pl.BlockSpec((1, 128), lambda i: (0, 0)),
        out_shape=jax.ShapeDtypeStruct((1, 128), jnp.float32),
        name="tc_bn_stats",
    )(partd, aggd)


def _e2_body(col_ref, row_ref, agg_ref, h_ref, st_ref, bn_ref, l1_ref, l2_ref,
             hp_ref, cw_ref, cb_ref, out_ref, acc_ref):
    i = pl.program_id(0)
    st = st_ref[0, :]
    s1c, s2c = st[0:1], st[1:2]
    s1r, s2r = st[2:3], st[3:4]
    m, zsum = st[4:5], st[5:6]
    muc = s1c / float(N * D)
    varc = s2c / float(N * D) - muc * muc
    mur = s1r / float(N * (K - 1))
    varr = s2r / float(N * (K - 1)) - mur * mur
    gr, br = bn_ref[0, 0:1], bn_ref[1, 0:1]
    gc, bc = bn_ref[2, 0:1], bn_ref[3, 0:1]
    ac = gc * lax.rsqrt(varc + 1e-5)
    bcs = bc - muc * ac
    ar = gr * lax.rsqrt(varr + 1e-5)
    brs = br - mur * ar

    coln = jnp.maximum(col_ref[...] * ac + bcs, 0.0)
    rown = jnp.maximum(row_ref[...] * ar + brs, 0.0)
    uf = (lax.dot_general(rown, l1_ref[...], (((1,), (0,)), ((), ())),
                          preferred_element_type=jnp.float32)
          + lax.dot_general(coln, l2_ref[...], (((1,), (0,)), ((), ())),
                            preferred_element_type=jnp.float32)
          + lax.dot_general(h_ref[...], hp_ref[...], (((1,), (1,)), ((), ())),
                            preferred_element_type=jnp.float32))
    uf = jnp.maximum(uf, 0.0)
    a = agg_ref[...]
    rr = lax.broadcasted_iota(jnp.int32, a.shape, 0)
    ll = lax.broadcasted_iota(jnp.int32, a.shape, 1)
    valid = (i * 8 + rr) * 128 + ll < N
    s = a * (float(K) ** -0.5)
    wblk = jnp.where(valid, jnp.exp(s - m), 0.0) / (zsum * float(N))
    part = jnp.zeros((1, 128), jnp.float32)
    for r in range(8):
        part = part + lax.dot_general(
            wblk[r:r + 1, :], uf[r * 128:(r + 1) * 128, :],
            (((1,), (0,)), ((), ())), preferred_element_type=jnp.float32)

    @pl.when(i == 0)
    def _():
        acc_ref[...] = jnp.zeros((1, 128), jnp.float32)

    acc_ref[...] += part

    @pl.when(i == (NPAD // E_B) - 1)
    def _():
        out_ref[...] = lax.dot_general(
            acc_ref[...], cw_ref[...], (((1,), (1,)), ((), ())),
            preferred_element_type=jnp.float32) + cb_ref[...]


def _final(col, row2d, aggd, h_pad, stats, bnvec, l1p, l2, hpw, cw, cb):
    return pl.pallas_call(
        _e2_body,
        grid=(NPAD // E_B,),
        in_specs=[
            pl.BlockSpec((E_B, D), lambda i: (i, 0)),
            pl.BlockSpec((E_B, 16), lambda i: (i, 0)),
            pl.BlockSpec((8, 128), lambda i: (i, 0)),
            pl.BlockSpec((E_B, D), lambda i: (i, 0)),
            pl.BlockSpec((1, 128), lambda i: (0, 0)),
            pl.BlockSpec((8, 128), lambda i: (0, 0)),
            pl.BlockSpec((16, D), lambda i: (0, 0)),
            pl.BlockSpec((D, D), lambda i: (0, 0)),
            pl.BlockSpec((D, D), lambda i: (0, 0)),
            pl.BlockSpec((NCLS, D), lambda i: (0, 0)),
            pl.BlockSpec((1, NCLS), lambda i: (0, 0)),
        ],
        out_specs=pl.BlockSpec((1, NCLS), lambda i: (0, 0)),
        out_shape=jax.ShapeDtypeStruct((1, NCLS), jnp.float32),
        scratch_shapes=[pltpu.VMEM((1, 128), jnp.float32)],
        name="tc_final",
    )(col, row2d, aggd, h_pad, stats, bnvec, l1p, l2, hpw, cw, cb)


def kernel(h, src_idx, fc_w, attn_w, convrow_w, convrow_b, bn_row_g, bn_row_b,
           convcol_w, convcol_b, bn_col_g, bn_col_b, localw, h_proj_w,
           conv_w, conv_b, classify_w, classify_b):
    f32 = jnp.float32
    h_pad = jnp.concatenate([h, jnp.zeros((NPAD - N, D), f32)], axis=0)
    srcf = jnp.concatenate(
        [src_idx,
         jnp.full((NPAD - N, K), NPAD - 1, jnp.int32)], axis=0).reshape(-1)

    a_l = attn_w[0, :D]
    a_r = attn_w[0, D:]
    w0 = convrow_w[0, 0, 0, :]
    w1 = convrow_w[0, 0, 1, :]
    wc = convcol_w[0, 0, :, 0]
    cwv = conv_w[:, 0]
    pvec = jnp.concatenate(
        [jnp.stack([a_l, w0, w1, a_r, cwv], axis=0),
         jnp.zeros((3, D), f32)], axis=0)
    smalls = jnp.concatenate(
        [wc, convrow_b, jnp.zeros((5,), f32)], axis=0)
    bnvec = jnp.stack([
        jnp.broadcast_to(bn_row_g[0], (128,)),
        jnp.broadcast_to(bn_row_b[0], (128,)),
        jnp.broadcast_to(bn_col_g[0], (128,)),
        jnp.broadcast_to(bn_col_b[0], (128,)),
    ] + [jnp.zeros((128,), f32)] * 4, axis=0)
    l1p = jnp.concatenate(
        [localw[:K - 1, :], jnp.zeros((16 - (K - 1), D), f32)], axis=0)
    l2 = localw[K - 1:, :]
    cb = classify_b.reshape(1, NCLS)

    deg2 = _deg_hist(srcf).reshape(NC, NPAD)
    z, zpa, zpb, zr1d = _prep(h_pad, fc_w, pvec, deg2)
    col, row_f, agg, part = _edge_all(srcf, zpa, zpb, zr1d, z, smalls)
    aggd = agg.reshape(NPAD // 128, 128)
    stats = _bn_stats(part.reshape(NW * 16 // 128, 128), aggd)
    return _final(col, row_f.reshape(NPAD, 16), aggd, h_pad, stats, bnvec,
                  l1p, l2, h_proj_w, classify_w, cb)
